# baseline (speedup 1.0000x reference)
import functools

import jax
import jax.numpy as jnp
from jax import lax
from jax.experimental import pallas as pl
from jax.experimental.pallas import tpu as pltpu
from jax.experimental.pallas import tpu_sc as plsc

N = 10000
E = 320000
D = 128
D2 = 256
G = 64

NC = 2
NS = 16
NW = NC * NS
EP = 327680
EH = EP // 2
EPW = EP // NW
EC = 80
NCHUNK = EPW // EC
EPWH = EH // NW
ECH = 80
NCHUNKH = EPWH // ECH
BN = 1000
BE = 640
RPS = 632
RLAST = N - RPS * (NS - 1)
NACC = N + 8
EPS = EH // NS
ECS = 80
NCHUNK_S = EPS // ECS

_f32 = jnp.float32
_mesh = plsc.VectorSubcoreMesh(core_axis_name="c", subcore_axis_name="s",
                               num_cores=NC, num_subcores=NS)


def _embed_body(x_ref, pe_ref, wx_ref, wpe_ref, eb_ref, pw_ref, pb_ref, hp_ref):
    x = x_ref[...]
    pe = pe_ref[...]
    h = (jnp.dot(x, wx_ref[...], preferred_element_type=_f32, precision=lax.Precision.HIGHEST)
         + jnp.dot(pe, wpe_ref[...], preferred_element_type=_f32, precision=lax.Precision.HIGHEST) + eb_ref[...])
    p = jnp.dot(pe, pw_ref[...], preferred_element_type=_f32, precision=lax.Precision.HIGHEST) + pb_ref[...]
    hp_ref[...] = jnp.concatenate([h, p], axis=1)


def _embed(x, pe, wx, wpe, eb, pw, pb):
    nb = N // BN
    return pl.pallas_call(
        _embed_body,
        grid=(nb,),
        in_specs=[
            pl.BlockSpec((BN, D), lambda i: (i, 0)),
            pl.BlockSpec((BN, 24), lambda i: (i, 0)),
            pl.BlockSpec((D, D), lambda i: (0, 0)),
            pl.BlockSpec((24, D), lambda i: (0, 0)),
            pl.BlockSpec((1, D), lambda i: (0, 0)),
            pl.BlockSpec((24, D), lambda i: (0, 0)),
            pl.BlockSpec((1, D), lambda i: (0, 0)),
        ],
        out_specs=pl.BlockSpec((BN, D2), lambda i: (i, 0)),
        out_shape=jax.ShapeDtypeStruct((N, D2), _f32),
    )(x, pe, wx, wpe, eb, pw, pb)


def _tables_body(s, hp_ref, agma_ref, agmb_ref, agpa_ref, agpb_ref,
                 ws_ref, wr_ref, b_ref, hpn_ref, send_ref, rec_ref):
    hp = hp_ref[...]
    ag = jnp.concatenate([agma_ref[...] + agmb_ref[...],
                          agpa_ref[...] + agpb_ref[...]], axis=1)
    hpn = s * hp + ag
    hpn_ref[...] = hpn
    send_ref[...] = jnp.dot(hpn, ws_ref[...], preferred_element_type=_f32, precision=lax.Precision.HIGHEST) + b_ref[...]
    rec_ref[...] = jnp.dot(hpn, wr_ref[...], preferred_element_type=_f32, precision=lax.Precision.HIGHEST)


def _tables(hp, agma, agmb, agpa, agpb, ws, wr, b, s):
    nb = N // BN
    return pl.pallas_call(
        functools.partial(_tables_body, float(s)),
        grid=(nb,),
        in_specs=[
            pl.BlockSpec((BN, D2), lambda i: (i, 0)),
            pl.BlockSpec((BN, D), lambda i: (i, 0)),
            pl.BlockSpec((BN, D), lambda i: (i, 0)),
            pl.BlockSpec((BN, D), lambda i: (i, 0)),
            pl.BlockSpec((BN, D), lambda i: (i, 0)),
            pl.BlockSpec((D2, D2), lambda i: (0, 0)),
            pl.BlockSpec((D2, D2), lambda i: (0, 0)),
            pl.BlockSpec((1, D2), lambda i: (0, 0)),
        ],
        out_specs=[
            pl.BlockSpec((BN, D2), lambda i: (i, 0)),
            pl.BlockSpec((BN, D2), lambda i: (i, 0)),
            pl.BlockSpec((BN, D2), lambda i: (i, 0)),
        ],
        out_shape=[
            jax.ShapeDtypeStruct((N, D2), _f32),
            jax.ShapeDtypeStruct((N, D2), _f32),
            jax.ShapeDtypeStruct((N, D2), _f32),
        ],
    )(hp, agma, agmb, agpa, agpb, ws, wr, b)


def _silu(z):
    return z * jax.nn.sigmoid(z)


def _dist_body(qs_ref, qr_ref, dist_ref):
    diff = qs_ref[...] - qr_ref[...]
    dist_ref[...] = jnp.sqrt(jnp.sum(diff * diff, axis=1, keepdims=True) + 1e-12)


def _dist(pos_s, pos_r):
    nb = EP // BE
    return pl.pallas_call(
        _dist_body,
        grid=(nb,),
        in_specs=[
            pl.BlockSpec((BE, D), lambda i: (i, 0)),
            pl.BlockSpec((BE, D), lambda i: (i, 0)),
        ],
        out_specs=pl.BlockSpec((BE, 1), lambda i: (i, 0)),
        out_shape=jax.ShapeDtypeStruct((EP, 1), _f32),
    )(pos_s, pos_r)


def _edge_body(ps_ref, pr_ref, dist_ref, wd_ref, m2w_ref, m2b_ref,
               p2w_ref, p2b_ref, msgm_ref, msgp_ref):
    pre = ps_ref[...] + pr_ref[...] + dist_ref[...] * wd_ref[...]
    u = _silu(pre[:, :D])
    v = jnp.tanh(pre[:, D:])
    msgm_ref[...] = _silu(jnp.dot(u, m2w_ref[...], preferred_element_type=_f32, precision=lax.Precision.HIGHEST)
                          + m2b_ref[...])
    msgp_ref[...] = jnp.tanh(jnp.dot(v, p2w_ref[...], preferred_element_type=_f32, precision=lax.Precision.HIGHEST)
                             + p2b_ref[...])


def _edge_mlp(preS, preR, dist, wd, m2w, m2b, p2w, p2b):
    nb = EH // BE
    return pl.pallas_call(
        _edge_body,
        grid=(nb,),
        in_specs=[
            pl.BlockSpec((BE, D2), lambda i: (i, 0)),
            pl.BlockSpec((BE, D2), lambda i: (i, 0)),
            pl.BlockSpec((BE, 1), lambda i: (i, 0)),
            pl.BlockSpec((1, D2), lambda i: (0, 0)),
            pl.BlockSpec((D, D), lambda i: (0, 0)),
            pl.BlockSpec((1, D), lambda i: (0, 0)),
            pl.BlockSpec((D, D), lambda i: (0, 0)),
            pl.BlockSpec((1, D), lambda i: (0, 0)),
        ],
        out_specs=[
            pl.BlockSpec((BE, D), lambda i: (i, 0)),
            pl.BlockSpec((BE, D), lambda i: (i, 0)),
        ],
        out_shape=[
            jax.ShapeDtypeStruct((EH, D), _f32),
            jax.ShapeDtypeStruct((EH, D), _f32),
        ],
    )(preS, preR, dist, wd, m2w, m2b, p2w, p2b)


def _readout_body(hp_ref, agma_ref, agmb_ref, b3_ref, w1_ref, b1_ref, w2_ref,
                  b2_ref, r1_ref, rb1_ref, r2_ref, rb2_ref, out_ref, pool_ref):
    i = pl.program_id(0)
    nb = pl.num_programs(0)

    @pl.when(i == 0)
    def _init():
        pool_ref[...] = jnp.zeros_like(pool_ref)

    h = 2.0 * hp_ref[:, :D] + agma_ref[...] + agmb_ref[...]
    q = _silu(jnp.dot(h, w1_ref[...], preferred_element_type=_f32, precision=lax.Precision.HIGHEST) + b1_ref[...])
    q = jnp.dot(q, w2_ref[...], preferred_element_type=_f32, precision=lax.Precision.HIGHEST) + b2_ref[...]
    brow = b3_ref[0]
    gid = lax.broadcasted_iota(jnp.int32, (G, BN), 0)
    onehot = (brow == gid).astype(_f32)
    pool_ref[...] += jnp.dot(onehot, q, preferred_element_type=_f32, precision=lax.Precision.HIGHEST)

    @pl.when(i == nb - 1)
    def _fin():
        g = pool_ref[...]
        o = _silu(jnp.dot(g, r1_ref[...], preferred_element_type=_f32, precision=lax.Precision.HIGHEST) + rb1_ref[...])
        out_ref[...] = jnp.dot(o, r2_ref[...], preferred_element_type=_f32, precision=lax.Precision.HIGHEST) + rb2_ref[...]


def _readout(hp, agma, agmb, batch3, w1, b1, w2, b2, r1, rb1, r2p, rb2p):
    nb = N // BN
    return pl.pallas_call(
        _readout_body,
        grid=(nb,),
        in_specs=[
            pl.BlockSpec((BN, D2), lambda i: (i, 0)),
            pl.BlockSpec((BN, D), lambda i: (i, 0)),
            pl.BlockSpec((BN, D), lambda i: (i, 0)),
            pl.BlockSpec((1, 1, BN), lambda i: (i, 0, 0)),
            pl.BlockSpec((D, D), lambda i: (0, 0)),
            pl.BlockSpec((1, D), lambda i: (0, 0)),
            pl.BlockSpec((D, D), lambda i: (0, 0)),
            pl.BlockSpec((1, D), lambda i: (0, 0)),
            pl.BlockSpec((D, D), lambda i: (0, 0)),
            pl.BlockSpec((1, D), lambda i: (0, 0)),
            pl.BlockSpec((D, D), lambda i: (0, 0)),
            pl.BlockSpec((1, D), lambda i: (0, 0)),
        ],
        out_specs=pl.BlockSpec((G, D), lambda i: (0, 0)),
        out_shape=jax.ShapeDtypeStruct((G, D), _f32),
        scratch_shapes=[pltpu.VMEM((G, D), _f32)],
    )(hp, agma, agmb, batch3, w1, b1, w2, b2, r1, rb1, r2p, rb2p)


@functools.partial(
    pl.kernel,
    out_type=[
        jax.ShapeDtypeStruct((EP, D), _f32),
        jax.ShapeDtypeStruct((EP, D), _f32),
    ],
    mesh=_mesh,
    scratch_types=[
        pltpu.VMEM((EC,), jnp.int32),
        pltpu.VMEM((EC,), jnp.int32),
        pltpu.VMEM((EC, D), _f32),
        pltpu.VMEM((EC, D), _f32),
        pltpu.SemaphoreType.DMA,
        pltpu.SemaphoreType.DMA,
    ],
)
def _pos_gather(send_hbm, rec_hbm, pos_hbm, outs_hbm, outr_hbm,
                sidx, ridx, bufs, bufr, sems, semr):
    wid = lax.axis_index("s") * NC + lax.axis_index("c")
    base = wid * EPW

    def body(i, carry):
        off = pl.multiple_of(base + i * EC, 8)
        pltpu.sync_copy(send_hbm.at[pl.ds(off, EC)], sidx)
        pltpu.sync_copy(rec_hbm.at[pl.ds(off, EC)], ridx)
        cs = pltpu.async_copy(pos_hbm.at[sidx], bufs, sems)
        cr = pltpu.async_copy(pos_hbm.at[ridx], bufr, semr)
        cs.wait()
        cr.wait()
        pltpu.sync_copy(bufs, outs_hbm.at[pl.ds(off, EC)])
        pltpu.sync_copy(bufr, outr_hbm.at[pl.ds(off, EC)])
        return carry

    lax.fori_loop(0, NCHUNK, body, 0)


@functools.partial(
    pl.kernel,
    out_type=[
        jax.ShapeDtypeStruct((EH, D2), _f32),
        jax.ShapeDtypeStruct((EH, D2), _f32),
    ],
    mesh=_mesh,
    scratch_types=[
        pltpu.VMEM((2, ECH), jnp.int32),
        pltpu.VMEM((2, ECH), jnp.int32),
        pltpu.VMEM((2, ECH, D2), _f32),
        pltpu.VMEM((2, ECH, D2), _f32),
        pltpu.SemaphoreType.DMA((2,)),
        pltpu.SemaphoreType.DMA((2,)),
        pltpu.SemaphoreType.DMA((2,)),
    ],
)
def _edge_gather(send_hbm, rec_hbm, stab_hbm, rtab_hbm, outs_hbm, outr_hbm,
                 sidx, ridx, bufS, bufR, semi, semg, semw):
    wid = lax.axis_index("s") * NC + lax.axis_index("c")
    base = wid * EPWH

    def idx_off(i):
        return pl.multiple_of(base + i * ECH, 8)

    def start_idx(i, b):
        off = idx_off(i)
        pltpu.async_copy(send_hbm.at[pl.ds(off, ECH)], sidx.at[b], semi.at[b])
        pltpu.async_copy(rec_hbm.at[pl.ds(off, ECH)], ridx.at[b], semi.at[b])

    def wait_idx(i, b):
        off = idx_off(i)
        pltpu.make_async_copy(send_hbm.at[pl.ds(off, ECH)], sidx.at[b], semi.at[b]).wait()
        pltpu.make_async_copy(rec_hbm.at[pl.ds(off, ECH)], ridx.at[b], semi.at[b]).wait()

    start_idx(0, 0)
    start_idx(1, 1)

    def pair(k, carry):
        descs = []
        for b in range(2):
            i = k + b
            wait_idx(i, b)
            descs.append(
                (pltpu.async_copy(stab_hbm.at[sidx.at[b]], bufS.at[b], semg.at[b]),
                 pltpu.async_copy(rtab_hbm.at[ridx.at[b]], bufR.at[b], semg.at[b])))
        wdescs = []
        for b in range(2):
            i = k + b
            descs[b][0].wait()
            descs[b][1].wait()
            nxt = jnp.minimum(i + 2, NCHUNKH - 1)
            start_idx(nxt, b)
            off = idx_off(i)
            wdescs.append(
                (pltpu.async_copy(bufS.at[b], outs_hbm.at[pl.ds(off, ECH)], semw.at[b]),
                 pltpu.async_copy(bufR.at[b], outr_hbm.at[pl.ds(off, ECH)], semw.at[b])))
        for b in range(2):
            wdescs[b][0].wait()
            wdescs[b][1].wait()
        return carry

    lax.fori_loop(0, NCHUNKH // 2, lambda k, c: pair(2 * k, c), 0)

    wait_idx(NCHUNKH - 1, 0)
    wait_idx(NCHUNKH - 1, 1)


@functools.partial(
    pl.kernel,
    out_type=[
        jax.ShapeDtypeStruct((N, D), _f32),
        jax.ShapeDtypeStruct((N, D), _f32),
    ],
    mesh=_mesh,
    scratch_types=[
        pltpu.VMEM((2, ECS), jnp.int32),
        pltpu.VMEM((2, ECS, D), _f32),
        pltpu.VMEM_SHARED((NACC, D), _f32),
        pltpu.SemaphoreType.DMA((2,)),
        pltpu.SemaphoreType.DMA((2,)),
    ],
)
def _scatter_add(msgm_hbm, msgp_hbm, rec_hbm, zero_hbm, outm_hbm, outp_hbm,
                 ridx, mbuf, acc, semL, semS):
    c = lax.axis_index("c")
    s = lax.axis_index("s")
    rbase = pl.multiple_of(s * RPS, 8)

    def _init(nrows):
        pltpu.sync_copy(zero_hbm.at[pl.ds(rbase, nrows)], acc.at[pl.ds(rbase, nrows)])

    @pl.when(s < NS - 1)
    def _i0():
        _init(RPS)

    @pl.when(s == NS - 1)
    def _i1():
        _init(RLAST)

    plsc.subcore_barrier()

    def run(msg_hbm, out_hbm):
        ebase = s * EPS

        def chunk_off(i):
            return pl.multiple_of(ebase + i * ECS, 8)

        def start_load(i, b):
            off = chunk_off(i)
            pltpu.async_copy(rec_hbm.at[pl.ds(off, ECS)], ridx.at[b], semL.at[b])
            pltpu.async_copy(msg_hbm.at[pl.ds(off, ECS)], mbuf.at[b], semL.at[b])

        def wait_load(i, b):
            off = chunk_off(i)
            pltpu.make_async_copy(rec_hbm.at[pl.ds(off, ECS)], ridx.at[b], semL.at[b]).wait()
            pltpu.make_async_copy(msg_hbm.at[pl.ds(off, ECS)], mbuf.at[b], semL.at[b]).wait()

        start_load(0, 0)
        start_load(1, 1)

        def pair(k, carry):
            sdescs = []
            for b in range(2):
                i = k + b
                wait_load(i, b)
                sdescs.append(pltpu.async_copy(
                    mbuf.at[b], acc.at[ridx.at[b]], semS.at[b], add=True))
            for b in range(2):
                i = k + b
                sdescs[b].wait()
                nxt = jnp.minimum(i + 2, NCHUNK_S - 1)
                start_load(nxt, b)
            return carry

        lax.fori_loop(0, NCHUNK_S // 2, lambda k, c: pair(2 * k, c), 0)
        wait_load(NCHUNK_S - 1, 0)
        wait_load(NCHUNK_S - 1, 1)
        plsc.subcore_barrier()

        def _fin(nrows):
            pltpu.sync_copy(acc.at[pl.ds(rbase, nrows)], out_hbm.at[pl.ds(rbase, nrows)])

        @pl.when(s < NS - 1)
        def _f0():
            _fin(RPS)

        @pl.when(s == NS - 1)
        def _f1():
            _fin(RLAST)

    @pl.when(c == 0)
    def _c0():
        run(msgm_hbm, outm_hbm)

    @pl.when(c == 1)
    def _c1():
        run(msgp_hbm, outp_hbm)


def kernel(x, pos, pe, edge_index, batch,
           embed_W, embed_b, pe_W, pe_b,
           m1_W, m1_b, m2_W, m2_b,
           p1_W, p1_b, p2_W, p2_b,
           pr1_W, pr1_b, pr2_W, pr2_b,
           r1_W, r1_b, r2_W, r2_b):
    L = m1_W.shape[0]
    send = edge_index[0]
    rec = edge_index[1]
    pos16 = jnp.zeros((N, D), _f32).at[:, :3].set(pos)
    zero_nd = jnp.zeros((N, D), _f32)
    batch3 = batch.astype(jnp.int32).reshape(N // BN, 1, BN)

    row = lambda v: v.reshape(1, -1)
    npad = EP - E
    pad_g = (jnp.arange(npad, dtype=jnp.int32) * 37) % N
    pad_s = N + (jnp.arange(npad, dtype=jnp.int32) % 8)
    send_p = jnp.concatenate([send, pad_g])
    rec_gp = jnp.concatenate([rec, pad_g])
    rec_sp = jnp.concatenate([rec, pad_s])
    sendA, sendB = send_p[:EH], send_p[EH:]
    recA, recB = rec_gp[:EH], rec_gp[EH:]
    recsA, recsB = rec_sp[:EH], rec_sp[EH:]
    hp = _embed(x, pe, embed_W[:D], embed_W[D:], row(embed_b), pe_W, row(pe_b))
    pos_s, pos_r = _pos_gather(send_p, rec_gp, pos16)
    dist = _dist(pos_s, pos_r)
    distA, distB = dist[:EH], dist[EH:]

    Z = jnp.zeros((D, D), _f32)
    agmA = agmB = agpA = agpB = zero_nd
    for l in range(L):
        ws = jnp.concatenate([
            jnp.concatenate([m1_W[l, 0:D], Z], axis=1),
            jnp.concatenate([m1_W[l, D:2 * D], p1_W[l, 0:D]], axis=1)], axis=0)
        wr = jnp.concatenate([
            jnp.concatenate([m1_W[l, 2 * D:3 * D], Z], axis=1),
            jnp.concatenate([m1_W[l, 3 * D:4 * D], p1_W[l, D:2 * D]], axis=1)], axis=0)
        bias = jnp.concatenate([m1_b[l], p1_b[l]]).reshape(1, D2)
        wd = jnp.concatenate([m1_W[l, 4 * D], p1_W[l, 2 * D]]).reshape(1, D2)

        hp, stab, rtab = _tables(hp, agmA, agmB, agpA, agpB, ws, wr, bias,
                                 1 if l == 0 else 2)
        preSA, preRA = _edge_gather(sendA, recA, stab, rtab)
        msgmA, msgpA = _edge_mlp(preSA, preRA, distA, wd,
                                 m2_W[l], row(m2_b[l]), p2_W[l], row(p2_b[l]))
        preSB, preRB = _edge_gather(sendB, recB, stab, rtab)
        msgmB, msgpB = _edge_mlp(preSB, preRB, distB, wd,
                                 m2_W[l], row(m2_b[l]), p2_W[l], row(p2_b[l]))
        agmA, agpA = _scatter_add(msgmA, msgpA, recsA, zero_nd)
        agmB, agpB = _scatter_add(msgmB, msgpB, recsB, zero_nd)

    r2p = jnp.zeros((D, D), _f32).at[:, :1].set(r2_W)
    rb2p = jnp.zeros((1, D), _f32).at[0, 0].set(r2_b[0])
    out = _readout(hp, agmA, agmB, batch3, pr1_W, row(pr1_b), pr2_W, row(pr2_b),
                   r1_W, row(r1_b), r2p, rb2p)
    return out[:, 0]


# speedup vs baseline: 1.0130x; 1.0130x over previous
import functools

import jax
import jax.numpy as jnp
from jax import lax
from jax.experimental import pallas as pl
from jax.experimental.pallas import tpu as pltpu
from jax.experimental.pallas import tpu_sc as plsc

N = 10000
E = 320000
D = 128
D2 = 256
G = 64

NC = 2
NS = 16
NW = NC * NS
EP = 327680
EH = EP // 2
EPW = EP // NW
EC = 128
NCHUNK = EPW // EC
EPWH = EH // NW
ECH = 80
NCHUNKH = EPWH // ECH
BN = 1000
BE = 640
RPS = 632
RLAST = N - RPS * (NS - 1)
NACC = N + 8
EPS = EH // NS
ECS = 128
NCHUNK_S = EPS // ECS

_f32 = jnp.float32
_mesh = plsc.VectorSubcoreMesh(core_axis_name="c", subcore_axis_name="s",
                               num_cores=NC, num_subcores=NS)


def _embed_body(x_ref, pe_ref, wx_ref, wpe_ref, eb_ref, pw_ref, pb_ref, hp_ref):
    x = x_ref[...]
    pe = pe_ref[...]
    h = (jnp.dot(x, wx_ref[...], preferred_element_type=_f32, precision=lax.Precision.HIGHEST)
         + jnp.dot(pe, wpe_ref[...], preferred_element_type=_f32, precision=lax.Precision.HIGHEST) + eb_ref[...])
    p = jnp.dot(pe, pw_ref[...], preferred_element_type=_f32, precision=lax.Precision.HIGHEST) + pb_ref[...]
    hp_ref[...] = jnp.concatenate([h, p], axis=1)


def _embed(x, pe, wx, wpe, eb, pw, pb):
    nb = N // BN
    return pl.pallas_call(
        _embed_body,
        grid=(nb,),
        in_specs=[
            pl.BlockSpec((BN, D), lambda i: (i, 0)),
            pl.BlockSpec((BN, 24), lambda i: (i, 0)),
            pl.BlockSpec((D, D), lambda i: (0, 0)),
            pl.BlockSpec((24, D), lambda i: (0, 0)),
            pl.BlockSpec((1, D), lambda i: (0, 0)),
            pl.BlockSpec((24, D), lambda i: (0, 0)),
            pl.BlockSpec((1, D), lambda i: (0, 0)),
        ],
        out_specs=pl.BlockSpec((BN, D2), lambda i: (i, 0)),
        out_shape=jax.ShapeDtypeStruct((N, D2), _f32),
    )(x, pe, wx, wpe, eb, pw, pb)


def _tables_body(s, hp_ref, agma_ref, agmb_ref, agpa_ref, agpb_ref,
                 ws_ref, wr_ref, b_ref, hpn_ref, send_ref, rec_ref):
    hp = hp_ref[...]
    ag = jnp.concatenate([agma_ref[...] + agmb_ref[...],
                          agpa_ref[...] + agpb_ref[...]], axis=1)
    hpn = s * hp + ag
    hpn_ref[...] = hpn
    send_ref[...] = jnp.dot(hpn, ws_ref[...], preferred_element_type=_f32, precision=lax.Precision.HIGHEST) + b_ref[...]
    rec_ref[...] = jnp.dot(hpn, wr_ref[...], preferred_element_type=_f32, precision=lax.Precision.HIGHEST)


def _tables(hp, agma, agmb, agpa, agpb, ws, wr, b, s):
    nb = N // BN
    return pl.pallas_call(
        functools.partial(_tables_body, float(s)),
        grid=(nb,),
        in_specs=[
            pl.BlockSpec((BN, D2), lambda i: (i, 0)),
            pl.BlockSpec((BN, D), lambda i: (i, 0)),
            pl.BlockSpec((BN, D), lambda i: (i, 0)),
            pl.BlockSpec((BN, D), lambda i: (i, 0)),
            pl.BlockSpec((BN, D), lambda i: (i, 0)),
            pl.BlockSpec((D2, D2), lambda i: (0, 0)),
            pl.BlockSpec((D2, D2), lambda i: (0, 0)),
            pl.BlockSpec((1, D2), lambda i: (0, 0)),
        ],
        out_specs=[
            pl.BlockSpec((BN, D2), lambda i: (i, 0)),
            pl.BlockSpec((BN, D2), lambda i: (i, 0)),
            pl.BlockSpec((BN, D2), lambda i: (i, 0)),
        ],
        out_shape=[
            jax.ShapeDtypeStruct((N, D2), _f32),
            jax.ShapeDtypeStruct((N, D2), _f32),
            jax.ShapeDtypeStruct((N, D2), _f32),
        ],
    )(hp, agma, agmb, agpa, agpb, ws, wr, b)


def _silu(z):
    return z * jax.nn.sigmoid(z)


def _dist_body(qs_ref, qr_ref, dist_ref):
    diff = qs_ref[...] - qr_ref[...]
    dist_ref[...] = jnp.sqrt(jnp.sum(diff * diff, axis=1, keepdims=True) + 1e-12)


def _dist(pos_s, pos_r):
    nb = EP // BE
    return pl.pallas_call(
        _dist_body,
        grid=(nb,),
        in_specs=[
            pl.BlockSpec((BE, 16), lambda i: (i, 0)),
            pl.BlockSpec((BE, 16), lambda i: (i, 0)),
        ],
        out_specs=pl.BlockSpec((BE, 1), lambda i: (i, 0)),
        out_shape=jax.ShapeDtypeStruct((EP, 1), _f32),
    )(pos_s, pos_r)


def _edge_body(ps_ref, pr_ref, dist_ref, wd_ref, m2w_ref, m2b_ref,
               p2w_ref, p2b_ref, msgm_ref, msgp_ref):
    pre = ps_ref[...] + pr_ref[...] + dist_ref[...] * wd_ref[...]
    u = _silu(pre[:, :D])
    v = jnp.tanh(pre[:, D:])
    msgm_ref[...] = _silu(jnp.dot(u, m2w_ref[...], preferred_element_type=_f32, precision=lax.Precision.HIGHEST)
                          + m2b_ref[...])
    msgp_ref[...] = jnp.tanh(jnp.dot(v, p2w_ref[...], preferred_element_type=_f32, precision=lax.Precision.HIGHEST)
                             + p2b_ref[...])


def _edge_mlp(preS, preR, dist, wd, m2w, m2b, p2w, p2b):
    nb = EH // BE
    return pl.pallas_call(
        _edge_body,
        grid=(nb,),
        in_specs=[
            pl.BlockSpec((BE, D2), lambda i: (i, 0)),
            pl.BlockSpec((BE, D2), lambda i: (i, 0)),
            pl.BlockSpec((BE, 1), lambda i: (i, 0)),
            pl.BlockSpec((1, D2), lambda i: (0, 0)),
            pl.BlockSpec((D, D), lambda i: (0, 0)),
            pl.BlockSpec((1, D), lambda i: (0, 0)),
            pl.BlockSpec((D, D), lambda i: (0, 0)),
            pl.BlockSpec((1, D), lambda i: (0, 0)),
        ],
        out_specs=[
            pl.BlockSpec((BE, D), lambda i: (i, 0)),
            pl.BlockSpec((BE, D), lambda i: (i, 0)),
        ],
        out_shape=[
            jax.ShapeDtypeStruct((EH, D), _f32),
            jax.ShapeDtypeStruct((EH, D), _f32),
        ],
    )(preS, preR, dist, wd, m2w, m2b, p2w, p2b)


def _readout_body(hp_ref, agma_ref, agmb_ref, b3_ref, w1_ref, b1_ref, w2_ref,
                  b2_ref, r1_ref, rb1_ref, r2_ref, rb2_ref, out_ref, pool_ref):
    i = pl.program_id(0)
    nb = pl.num_programs(0)

    @pl.when(i == 0)
    def _init():
        pool_ref[...] = jnp.zeros_like(pool_ref)

    h = 2.0 * hp_ref[:, :D] + agma_ref[...] + agmb_ref[...]
    q = _silu(jnp.dot(h, w1_ref[...], preferred_element_type=_f32, precision=lax.Precision.HIGHEST) + b1_ref[...])
    q = jnp.dot(q, w2_ref[...], preferred_element_type=_f32, precision=lax.Precision.HIGHEST) + b2_ref[...]
    brow = b3_ref[0]
    gid = lax.broadcasted_iota(jnp.int32, (G, BN), 0)
    onehot = (brow == gid).astype(_f32)
    pool_ref[...] += jnp.dot(onehot, q, preferred_element_type=_f32, precision=lax.Precision.HIGHEST)

    @pl.when(i == nb - 1)
    def _fin():
        g = pool_ref[...]
        o = _silu(jnp.dot(g, r1_ref[...], preferred_element_type=_f32, precision=lax.Precision.HIGHEST) + rb1_ref[...])
        out_ref[...] = jnp.dot(o, r2_ref[...], preferred_element_type=_f32, precision=lax.Precision.HIGHEST) + rb2_ref[...]


def _readout(hp, agma, agmb, batch3, w1, b1, w2, b2, r1, rb1, r2p, rb2p):
    nb = N // BN
    return pl.pallas_call(
        _readout_body,
        grid=(nb,),
        in_specs=[
            pl.BlockSpec((BN, D2), lambda i: (i, 0)),
            pl.BlockSpec((BN, D), lambda i: (i, 0)),
            pl.BlockSpec((BN, D), lambda i: (i, 0)),
            pl.BlockSpec((1, 1, BN), lambda i: (i, 0, 0)),
            pl.BlockSpec((D, D), lambda i: (0, 0)),
            pl.BlockSpec((1, D), lambda i: (0, 0)),
            pl.BlockSpec((D, D), lambda i: (0, 0)),
            pl.BlockSpec((1, D), lambda i: (0, 0)),
            pl.BlockSpec((D, D), lambda i: (0, 0)),
            pl.BlockSpec((1, D), lambda i: (0, 0)),
            pl.BlockSpec((D, D), lambda i: (0, 0)),
            pl.BlockSpec((1, D), lambda i: (0, 0)),
        ],
        out_specs=pl.BlockSpec((G, D), lambda i: (0, 0)),
        out_shape=jax.ShapeDtypeStruct((G, D), _f32),
        scratch_shapes=[pltpu.VMEM((G, D), _f32)],
    )(hp, agma, agmb, batch3, w1, b1, w2, b2, r1, rb1, r2p, rb2p)


@functools.partial(
    pl.kernel,
    out_type=[
        jax.ShapeDtypeStruct((EP, 16), _f32),
        jax.ShapeDtypeStruct((EP, 16), _f32),
    ],
    mesh=_mesh,
    scratch_types=[
        pltpu.VMEM((EC,), jnp.int32),
        pltpu.VMEM((EC,), jnp.int32),
        pltpu.VMEM((EC, 16), _f32),
        pltpu.VMEM((EC, 16), _f32),
        pltpu.SemaphoreType.DMA,
        pltpu.SemaphoreType.DMA,
    ],
    compiler_params=pltpu.CompilerParams(use_tc_tiling_on_sc=False),
)
def _pos_gather(send_hbm, rec_hbm, pos_hbm, outs_hbm, outr_hbm,
                sidx, ridx, bufs, bufr, sems, semr):
    wid = lax.axis_index("s") * NC + lax.axis_index("c")
    base = wid * EPW

    def body(i, carry):
        off = pl.multiple_of(base + i * EC, 8)
        pltpu.sync_copy(send_hbm.at[pl.ds(off, EC)], sidx)
        pltpu.sync_copy(rec_hbm.at[pl.ds(off, EC)], ridx)
        cs = pltpu.async_copy(pos_hbm.at[sidx], bufs, sems)
        cr = pltpu.async_copy(pos_hbm.at[ridx], bufr, semr)
        cs.wait()
        cr.wait()
        pltpu.sync_copy(bufs, outs_hbm.at[pl.ds(off, EC)])
        pltpu.sync_copy(bufr, outr_hbm.at[pl.ds(off, EC)])
        return carry

    lax.fori_loop(0, NCHUNK, body, 0)


@functools.partial(
    pl.kernel,
    out_type=[
        jax.ShapeDtypeStruct((EH, D2), _f32),
        jax.ShapeDtypeStruct((EH, D2), _f32),
    ],
    mesh=_mesh,
    scratch_types=[
        pltpu.VMEM((2, ECH), jnp.int32),
        pltpu.VMEM((2, ECH), jnp.int32),
        pltpu.VMEM((2, ECH, D2), _f32),
        pltpu.VMEM((2, ECH, D2), _f32),
        pltpu.SemaphoreType.DMA((2,)),
        pltpu.SemaphoreType.DMA((2,)),
        pltpu.SemaphoreType.DMA((2,)),
    ],
)
def _edge_gather(send_hbm, rec_hbm, stab_hbm, rtab_hbm, outs_hbm, outr_hbm,
                 sidx, ridx, bufS, bufR, semi, semg, semw):
    wid = lax.axis_index("s") * NC + lax.axis_index("c")
    base = wid * EPWH

    def idx_off(i):
        return pl.multiple_of(base + i * ECH, 8)

    def start_idx(i, b):
        off = idx_off(i)
        pltpu.async_copy(send_hbm.at[pl.ds(off, ECH)], sidx.at[b], semi.at[b])
        pltpu.async_copy(rec_hbm.at[pl.ds(off, ECH)], ridx.at[b], semi.at[b])

    def wait_idx(i, b):
        off = idx_off(i)
        pltpu.make_async_copy(send_hbm.at[pl.ds(off, ECH)], sidx.at[b], semi.at[b]).wait()
        pltpu.make_async_copy(rec_hbm.at[pl.ds(off, ECH)], ridx.at[b], semi.at[b]).wait()

    start_idx(0, 0)
    start_idx(1, 1)

    def pair(k, carry):
        descs = []
        for b in range(2):
            i = k + b
            wait_idx(i, b)
            descs.append(
                (pltpu.async_copy(stab_hbm.at[sidx.at[b]], bufS.at[b], semg.at[b]),
                 pltpu.async_copy(rtab_hbm.at[ridx.at[b]], bufR.at[b], semg.at[b])))
        wdescs = []
        for b in range(2):
            i = k + b
            descs[b][0].wait()
            descs[b][1].wait()
            nxt = jnp.minimum(i + 2, NCHUNKH - 1)
            start_idx(nxt, b)
            off = idx_off(i)
            wdescs.append(
                (pltpu.async_copy(bufS.at[b], outs_hbm.at[pl.ds(off, ECH)], semw.at[b]),
                 pltpu.async_copy(bufR.at[b], outr_hbm.at[pl.ds(off, ECH)], semw.at[b])))
        for b in range(2):
            wdescs[b][0].wait()
            wdescs[b][1].wait()
        return carry

    lax.fori_loop(0, NCHUNKH // 2, lambda k, c: pair(2 * k, c), 0)

    wait_idx(NCHUNKH - 1, 0)
    wait_idx(NCHUNKH - 1, 1)


@functools.partial(
    pl.kernel,
    out_type=[
        jax.ShapeDtypeStruct((N, D), _f32),
        jax.ShapeDtypeStruct((N, D), _f32),
    ],
    mesh=_mesh,
    scratch_types=[
        pltpu.VMEM((2, ECS), jnp.int32),
        pltpu.VMEM((2, ECS, D), _f32),
        pltpu.VMEM_SHARED((NACC, D), _f32),
        pltpu.SemaphoreType.DMA((2,)),
        pltpu.SemaphoreType.DMA((2,)),
    ],
)
def _scatter_add(msgm_hbm, msgp_hbm, rec_hbm, zero_hbm, outm_hbm, outp_hbm,
                 ridx, mbuf, acc, semL, semS):
    c = lax.axis_index("c")
    s = lax.axis_index("s")
    rbase = pl.multiple_of(s * RPS, 8)

    def _init(nrows):
        pltpu.sync_copy(zero_hbm.at[pl.ds(rbase, nrows)], acc.at[pl.ds(rbase, nrows)])

    @pl.when(s < NS - 1)
    def _i0():
        _init(RPS)

    @pl.when(s == NS - 1)
    def _i1():
        _init(RLAST)

    plsc.subcore_barrier()

    def run(msg_hbm, out_hbm):
        ebase = s * EPS

        def chunk_off(i):
            return pl.multiple_of(ebase + i * ECS, 8)

        def start_load(i, b):
            off = chunk_off(i)
            pltpu.async_copy(rec_hbm.at[pl.ds(off, ECS)], ridx.at[b], semL.at[b])
            pltpu.async_copy(msg_hbm.at[pl.ds(off, ECS)], mbuf.at[b], semL.at[b])

        def wait_load(i, b):
            off = chunk_off(i)
            pltpu.make_async_copy(rec_hbm.at[pl.ds(off, ECS)], ridx.at[b], semL.at[b]).wait()
            pltpu.make_async_copy(msg_hbm.at[pl.ds(off, ECS)], mbuf.at[b], semL.at[b]).wait()

        start_load(0, 0)
        start_load(1, 1)

        def pair(k, carry):
            sdescs = []
            for b in range(2):
                i = k + b
                wait_load(i, b)
                sdescs.append(pltpu.async_copy(
                    mbuf.at[b], acc.at[ridx.at[b]], semS.at[b], add=True))
            for b in range(2):
                i = k + b
                sdescs[b].wait()
                nxt = jnp.minimum(i + 2, NCHUNK_S - 1)
                start_load(nxt, b)
            return carry

        lax.fori_loop(0, NCHUNK_S // 2, lambda k, c: pair(2 * k, c), 0)
        wait_load(NCHUNK_S - 1, 0)
        wait_load(NCHUNK_S - 1, 1)
        plsc.subcore_barrier()

        def _fin(nrows):
            pltpu.sync_copy(acc.at[pl.ds(rbase, nrows)], out_hbm.at[pl.ds(rbase, nrows)])

        @pl.when(s < NS - 1)
        def _f0():
            _fin(RPS)

        @pl.when(s == NS - 1)
        def _f1():
            _fin(RLAST)

    @pl.when(c == 0)
    def _c0():
        run(msgm_hbm, outm_hbm)

    @pl.when(c == 1)
    def _c1():
        run(msgp_hbm, outp_hbm)


def kernel(x, pos, pe, edge_index, batch,
           embed_W, embed_b, pe_W, pe_b,
           m1_W, m1_b, m2_W, m2_b,
           p1_W, p1_b, p2_W, p2_b,
           pr1_W, pr1_b, pr2_W, pr2_b,
           r1_W, r1_b, r2_W, r2_b):
    L = m1_W.shape[0]
    send = edge_index[0]
    rec = edge_index[1]
    pos16 = jnp.zeros((N, 16), _f32).at[:, :3].set(pos)
    zero_nd = jnp.zeros((N, D), _f32)
    batch3 = batch.astype(jnp.int32).reshape(N // BN, 1, BN)

    row = lambda v: v.reshape(1, -1)
    npad = EP - E
    pad_g = (jnp.arange(npad, dtype=jnp.int32) * 37) % N
    pad_s = N + (jnp.arange(npad, dtype=jnp.int32) % 8)
    send_p = jnp.concatenate([send, pad_g])
    rec_gp = jnp.concatenate([rec, pad_g])
    rec_sp = jnp.concatenate([rec, pad_s])
    sendA, sendB = send_p[:EH], send_p[EH:]
    recA, recB = rec_gp[:EH], rec_gp[EH:]
    recsA, recsB = rec_sp[:EH], rec_sp[EH:]
    hp = _embed(x, pe, embed_W[:D], embed_W[D:], row(embed_b), pe_W, row(pe_b))
    pos_s, pos_r = _pos_gather(send_p, rec_gp, pos16)
    dist = _dist(pos_s, pos_r)
    distA, distB = dist[:EH], dist[EH:]

    Z = jnp.zeros((D, D), _f32)
    agmA = agmB = agpA = agpB = zero_nd
    for l in range(L):
        ws = jnp.concatenate([
            jnp.concatenate([m1_W[l, 0:D], Z], axis=1),
            jnp.concatenate([m1_W[l, D:2 * D], p1_W[l, 0:D]], axis=1)], axis=0)
        wr = jnp.concatenate([
            jnp.concatenate([m1_W[l, 2 * D:3 * D], Z], axis=1),
            jnp.concatenate([m1_W[l, 3 * D:4 * D], p1_W[l, D:2 * D]], axis=1)], axis=0)
        bias = jnp.concatenate([m1_b[l], p1_b[l]]).reshape(1, D2)
        wd = jnp.concatenate([m1_W[l, 4 * D], p1_W[l, 2 * D]]).reshape(1, D2)

        hp, stab, rtab = _tables(hp, agmA, agmB, agpA, agpB, ws, wr, bias,
                                 1 if l == 0 else 2)
        preSA, preRA = _edge_gather(sendA, recA, stab, rtab)
        msgmA, msgpA = _edge_mlp(preSA, preRA, distA, wd,
                                 m2_W[l], row(m2_b[l]), p2_W[l], row(p2_b[l]))
        preSB, preRB = _edge_gather(sendB, recB, stab, rtab)
        msgmB, msgpB = _edge_mlp(preSB, preRB, distB, wd,
                                 m2_W[l], row(m2_b[l]), p2_W[l], row(p2_b[l]))
        agmA, agpA = _scatter_add(msgmA, msgpA, recsA, zero_nd)
        agmB, agpB = _scatter_add(msgmB, msgpB, recsB, zero_nd)

    r2p = jnp.zeros((D, D), _f32).at[:, :1].set(r2_W)
    rb2p = jnp.zeros((1, D), _f32).at[0, 0].set(r2_b[0])
    out = _readout(hp, agmA, agmB, batch3, pr1_W, row(pr1_b), pr2_W, row(pr2_b),
                   r1_W, row(r1_b), r2p, rb2p)
    return out[:, 0]


# speedup vs baseline: 1.1183x; 1.1039x over previous
import functools

import jax
import jax.numpy as jnp
from jax import lax
from jax.experimental import pallas as pl
from jax.experimental.pallas import tpu as pltpu
from jax.experimental.pallas import tpu_sc as plsc

N = 10000
E = 320000
D = 128
D2 = 256
G = 64

NC = 2
NS = 16
NW = NC * NS
EP = 327680
EH = EP // 2
EPW = EP // NW
EC = 128
NCHUNK = EPW // EC
EPWH = EH // NW
ECH = 80
NCHUNKH = EPWH // ECH
BN = 1000
BE = 640
RPS = 632
RLAST = N - RPS * (NS - 1)
NACC = N + 8
EPS = EH // NS
ECS = 128
NCHUNK_S = EPS // ECS

_f32 = jnp.float32
_mesh = plsc.VectorSubcoreMesh(core_axis_name="c", subcore_axis_name="s",
                               num_cores=NC, num_subcores=NS)


def _embed_body(x_ref, pe_ref, wx_ref, wpe_ref, eb_ref, pw_ref, pb_ref, hp_ref):
    x = x_ref[...]
    pe = pe_ref[...]
    h = (jnp.dot(x, wx_ref[...], preferred_element_type=_f32, precision=lax.Precision.HIGHEST)
         + jnp.dot(pe, wpe_ref[...], preferred_element_type=_f32, precision=lax.Precision.HIGHEST) + eb_ref[...])
    p = jnp.dot(pe, pw_ref[...], preferred_element_type=_f32, precision=lax.Precision.HIGHEST) + pb_ref[...]
    hp_ref[...] = jnp.concatenate([h, p], axis=1)


def _embed(x, pe, wx, wpe, eb, pw, pb):
    nb = N // BN
    return pl.pallas_call(
        _embed_body,
        grid=(nb,),
        in_specs=[
            pl.BlockSpec((BN, D), lambda i: (i, 0)),
            pl.BlockSpec((BN, 24), lambda i: (i, 0)),
            pl.BlockSpec((D, D), lambda i: (0, 0)),
            pl.BlockSpec((24, D), lambda i: (0, 0)),
            pl.BlockSpec((1, D), lambda i: (0, 0)),
            pl.BlockSpec((24, D), lambda i: (0, 0)),
            pl.BlockSpec((1, D), lambda i: (0, 0)),
        ],
        out_specs=pl.BlockSpec((BN, D2), lambda i: (i, 0)),
        out_shape=jax.ShapeDtypeStruct((N, D2), _f32),
    )(x, pe, wx, wpe, eb, pw, pb)


def _tables_body(s, hp_ref, agma_ref, agmb_ref, agpa_ref, agpb_ref,
                 ws_ref, wr_ref, b_ref, hpn_ref, send_ref, rec_ref):
    hp = hp_ref[...]
    ag = jnp.concatenate([agma_ref[...] + agmb_ref[...],
                          agpa_ref[...] + agpb_ref[...]], axis=1)
    hpn = s * hp + ag
    hpn_ref[...] = hpn
    send_ref[...] = jnp.dot(hpn, ws_ref[...], preferred_element_type=_f32, precision=lax.Precision.HIGHEST) + b_ref[...]
    rec_ref[...] = jnp.dot(hpn, wr_ref[...], preferred_element_type=_f32, precision=lax.Precision.HIGHEST)


def _tables(hp, agma, agmb, agpa, agpb, ws, wr, b, s):
    nb = N // BN
    return pl.pallas_call(
        functools.partial(_tables_body, float(s)),
        grid=(nb,),
        in_specs=[
            pl.BlockSpec((BN, D2), lambda i: (i, 0)),
            pl.BlockSpec((BN, D), lambda i: (i, 0)),
            pl.BlockSpec((BN, D), lambda i: (i, 0)),
            pl.BlockSpec((BN, D), lambda i: (i, 0)),
            pl.BlockSpec((BN, D), lambda i: (i, 0)),
            pl.BlockSpec((D2, D2), lambda i: (0, 0)),
            pl.BlockSpec((D2, D2), lambda i: (0, 0)),
            pl.BlockSpec((1, D2), lambda i: (0, 0)),
        ],
        out_specs=[
            pl.BlockSpec((BN, D2), lambda i: (i, 0)),
            pl.BlockSpec((BN, D2), lambda i: (i, 0)),
            pl.BlockSpec((BN, D2), lambda i: (i, 0)),
        ],
        out_shape=[
            jax.ShapeDtypeStruct((N, D2), _f32),
            jax.ShapeDtypeStruct((N, D2), _f32),
            jax.ShapeDtypeStruct((N, D2), _f32),
        ],
    )(hp, agma, agmb, agpa, agpb, ws, wr, b)


def _silu(z):
    return z * jax.nn.sigmoid(z)


def _dist_body(qs_ref, qr_ref, dist_ref):
    diff = qs_ref[...] - qr_ref[...]
    dist_ref[...] = jnp.sqrt(jnp.sum(diff * diff, axis=1, keepdims=True) + 1e-12)


def _dist(pos_s, pos_r):
    nb = EP // BE
    return pl.pallas_call(
        _dist_body,
        grid=(nb,),
        in_specs=[
            pl.BlockSpec((BE, 16), lambda i: (i, 0)),
            pl.BlockSpec((BE, 16), lambda i: (i, 0)),
        ],
        out_specs=pl.BlockSpec((BE, 1), lambda i: (i, 0)),
        out_shape=jax.ShapeDtypeStruct((EP, 1), _f32),
    )(pos_s, pos_r)


def _edge_body(ps_ref, dist_ref, wd_ref, m2w_ref, m2b_ref,
               p2w_ref, p2b_ref, msgm_ref, msgp_ref):
    pre = ps_ref[...] + dist_ref[...] * wd_ref[...]
    u = _silu(pre[:, :D])
    v = jnp.tanh(pre[:, D:])
    msgm_ref[...] = _silu(jnp.dot(u, m2w_ref[...], preferred_element_type=_f32, precision=lax.Precision.HIGHEST)
                          + m2b_ref[...])
    msgp_ref[...] = jnp.tanh(jnp.dot(v, p2w_ref[...], preferred_element_type=_f32, precision=lax.Precision.HIGHEST)
                             + p2b_ref[...])


def _edge_mlp(preSR, dist, wd, m2w, m2b, p2w, p2b):
    nb = EH // BE
    return pl.pallas_call(
        _edge_body,
        grid=(nb,),
        in_specs=[
            pl.BlockSpec((BE, D2), lambda i: (i, 0)),
            pl.BlockSpec((BE, 1), lambda i: (i, 0)),
            pl.BlockSpec((1, D2), lambda i: (0, 0)),
            pl.BlockSpec((D, D), lambda i: (0, 0)),
            pl.BlockSpec((1, D), lambda i: (0, 0)),
            pl.BlockSpec((D, D), lambda i: (0, 0)),
            pl.BlockSpec((1, D), lambda i: (0, 0)),
        ],
        out_specs=[
            pl.BlockSpec((BE, D), lambda i: (i, 0)),
            pl.BlockSpec((BE, D), lambda i: (i, 0)),
        ],
        out_shape=[
            jax.ShapeDtypeStruct((EH, D), _f32),
            jax.ShapeDtypeStruct((EH, D), _f32),
        ],
    )(preSR, dist, wd, m2w, m2b, p2w, p2b)


def _readout_body(hp_ref, agma_ref, agmb_ref, b3_ref, w1_ref, b1_ref, w2_ref,
                  b2_ref, r1_ref, rb1_ref, r2_ref, rb2_ref, out_ref, pool_ref):
    i = pl.program_id(0)
    nb = pl.num_programs(0)

    @pl.when(i == 0)
    def _init():
        pool_ref[...] = jnp.zeros_like(pool_ref)

    h = 2.0 * hp_ref[:, :D] + agma_ref[...] + agmb_ref[...]
    q = _silu(jnp.dot(h, w1_ref[...], preferred_element_type=_f32, precision=lax.Precision.HIGHEST) + b1_ref[...])
    q = jnp.dot(q, w2_ref[...], preferred_element_type=_f32, precision=lax.Precision.HIGHEST) + b2_ref[...]
    brow = b3_ref[0]
    gid = lax.broadcasted_iota(jnp.int32, (G, BN), 0)
    onehot = (brow == gid).astype(_f32)
    pool_ref[...] += jnp.dot(onehot, q, preferred_element_type=_f32, precision=lax.Precision.HIGHEST)

    @pl.when(i == nb - 1)
    def _fin():
        g = pool_ref[...]
        o = _silu(jnp.dot(g, r1_ref[...], preferred_element_type=_f32, precision=lax.Precision.HIGHEST) + rb1_ref[...])
        out_ref[...] = jnp.dot(o, r2_ref[...], preferred_element_type=_f32, precision=lax.Precision.HIGHEST) + rb2_ref[...]


def _readout(hp, agma, agmb, batch3, w1, b1, w2, b2, r1, rb1, r2p, rb2p):
    nb = N // BN
    return pl.pallas_call(
        _readout_body,
        grid=(nb,),
        in_specs=[
            pl.BlockSpec((BN, D2), lambda i: (i, 0)),
            pl.BlockSpec((BN, D), lambda i: (i, 0)),
            pl.BlockSpec((BN, D), lambda i: (i, 0)),
            pl.BlockSpec((1, 1, BN), lambda i: (i, 0, 0)),
            pl.BlockSpec((D, D), lambda i: (0, 0)),
            pl.BlockSpec((1, D), lambda i: (0, 0)),
            pl.BlockSpec((D, D), lambda i: (0, 0)),
            pl.BlockSpec((1, D), lambda i: (0, 0)),
            pl.BlockSpec((D, D), lambda i: (0, 0)),
            pl.BlockSpec((1, D), lambda i: (0, 0)),
            pl.BlockSpec((D, D), lambda i: (0, 0)),
            pl.BlockSpec((1, D), lambda i: (0, 0)),
        ],
        out_specs=pl.BlockSpec((G, D), lambda i: (0, 0)),
        out_shape=jax.ShapeDtypeStruct((G, D), _f32),
        scratch_shapes=[pltpu.VMEM((G, D), _f32)],
    )(hp, agma, agmb, batch3, w1, b1, w2, b2, r1, rb1, r2p, rb2p)


@functools.partial(
    pl.kernel,
    out_type=[
        jax.ShapeDtypeStruct((EP, 16), _f32),
        jax.ShapeDtypeStruct((EP, 16), _f32),
    ],
    mesh=_mesh,
    scratch_types=[
        pltpu.VMEM((EC,), jnp.int32),
        pltpu.VMEM((EC,), jnp.int32),
        pltpu.VMEM((EC, 16), _f32),
        pltpu.VMEM((EC, 16), _f32),
        pltpu.SemaphoreType.DMA,
        pltpu.SemaphoreType.DMA,
    ],
    compiler_params=pltpu.CompilerParams(use_tc_tiling_on_sc=False),
)
def _pos_gather(send_hbm, rec_hbm, pos_hbm, outs_hbm, outr_hbm,
                sidx, ridx, bufs, bufr, sems, semr):
    wid = lax.axis_index("s") * NC + lax.axis_index("c")
    base = wid * EPW

    def body(i, carry):
        off = pl.multiple_of(base + i * EC, 8)
        pltpu.sync_copy(send_hbm.at[pl.ds(off, EC)], sidx)
        pltpu.sync_copy(rec_hbm.at[pl.ds(off, EC)], ridx)
        cs = pltpu.async_copy(pos_hbm.at[sidx], bufs, sems)
        cr = pltpu.async_copy(pos_hbm.at[ridx], bufr, semr)
        cs.wait()
        cr.wait()
        pltpu.sync_copy(bufs, outs_hbm.at[pl.ds(off, EC)])
        pltpu.sync_copy(bufr, outr_hbm.at[pl.ds(off, EC)])
        return carry

    lax.fori_loop(0, NCHUNK, body, 0)


@functools.partial(
    pl.kernel,
    out_type=jax.ShapeDtypeStruct((EH, D2), _f32),
    mesh=_mesh,
    scratch_types=[
        pltpu.VMEM((2, ECH), jnp.int32),
        pltpu.VMEM((2, ECH), jnp.int32),
        pltpu.VMEM((2, ECH, D2), _f32),
        pltpu.VMEM((2, ECH, D2), _f32),
        pltpu.SemaphoreType.DMA((2,)),
        pltpu.SemaphoreType.DMA((2,)),
        pltpu.SemaphoreType.DMA((2,)),
    ],
)
def _edge_gather(send_hbm, rec_hbm, stab_hbm, rtab_hbm, outs_hbm,
                 sidx, ridx, bufS, bufR, semi, semg, semw):
    wid = lax.axis_index("s") * NC + lax.axis_index("c")
    base = wid * EPWH

    def idx_off(i):
        return pl.multiple_of(base + i * ECH, 8)

    def start_idx(i, b):
        off = idx_off(i)
        pltpu.async_copy(send_hbm.at[pl.ds(off, ECH)], sidx.at[b], semi.at[b])
        pltpu.async_copy(rec_hbm.at[pl.ds(off, ECH)], ridx.at[b], semi.at[b])

    def wait_idx(i, b):
        off = idx_off(i)
        pltpu.make_async_copy(send_hbm.at[pl.ds(off, ECH)], sidx.at[b], semi.at[b]).wait()
        pltpu.make_async_copy(rec_hbm.at[pl.ds(off, ECH)], ridx.at[b], semi.at[b]).wait()

    start_idx(0, 0)
    start_idx(1, 1)

    def pair(k, carry):
        descs = []
        for b in range(2):
            i = k + b
            wait_idx(i, b)
            descs.append(
                (pltpu.async_copy(stab_hbm.at[sidx.at[b]], bufS.at[b], semg.at[b]),
                 pltpu.async_copy(rtab_hbm.at[ridx.at[b]], bufR.at[b], semg.at[b])))
        wdescs = []
        for b in range(2):
            i = k + b
            descs[b][0].wait()
            descs[b][1].wait()
            nxt = jnp.minimum(i + 2, NCHUNKH - 1)
            start_idx(nxt, b)
            bs2 = bufS.at[b]
            br2 = bufR.at[b]

            def add_row(r, carry2, bs2=bs2, br2=br2):
                for kk in range(D2 // 16):
                    sl = pl.ds(kk * 16, 16)
                    bs2[r, sl] = bs2[r, sl] + br2[r, sl]
                return carry2

            lax.fori_loop(0, ECH, add_row, 0)
            off = idx_off(i)
            wdescs.append(
                pltpu.async_copy(bufS.at[b], outs_hbm.at[pl.ds(off, ECH)], semw.at[b]))
        for b in range(2):
            wdescs[b].wait()
        return carry

    lax.fori_loop(0, NCHUNKH // 2, lambda k, c: pair(2 * k, c), 0)

    wait_idx(NCHUNKH - 1, 0)
    wait_idx(NCHUNKH - 1, 1)


@functools.partial(
    pl.kernel,
    out_type=[
        jax.ShapeDtypeStruct((N, D), _f32),
        jax.ShapeDtypeStruct((N, D), _f32),
    ],
    mesh=_mesh,
    scratch_types=[
        pltpu.VMEM((2, ECS), jnp.int32),
        pltpu.VMEM((2, ECS, D), _f32),
        pltpu.VMEM_SHARED((NACC, D), _f32),
        pltpu.SemaphoreType.DMA((2,)),
        pltpu.SemaphoreType.DMA((2,)),
    ],
)
def _scatter_add(msgm_hbm, msgp_hbm, rec_hbm, zero_hbm, outm_hbm, outp_hbm,
                 ridx, mbuf, acc, semL, semS):
    c = lax.axis_index("c")
    s = lax.axis_index("s")
    rbase = pl.multiple_of(s * RPS, 8)

    def _init(nrows):
        pltpu.sync_copy(zero_hbm.at[pl.ds(rbase, nrows)], acc.at[pl.ds(rbase, nrows)])

    @pl.when(s < NS - 1)
    def _i0():
        _init(RPS)

    @pl.when(s == NS - 1)
    def _i1():
        _init(RLAST)

    plsc.subcore_barrier()

    def run(msg_hbm, out_hbm):
        ebase = s * EPS

        def chunk_off(i):
            return pl.multiple_of(ebase + i * ECS, 8)

        def start_load(i, b):
            off = chunk_off(i)
            pltpu.async_copy(rec_hbm.at[pl.ds(off, ECS)], ridx.at[b], semL.at[b])
            pltpu.async_copy(msg_hbm.at[pl.ds(off, ECS)], mbuf.at[b], semL.at[b])

        def wait_load(i, b):
            off = chunk_off(i)
            pltpu.make_async_copy(rec_hbm.at[pl.ds(off, ECS)], ridx.at[b], semL.at[b]).wait()
            pltpu.make_async_copy(msg_hbm.at[pl.ds(off, ECS)], mbuf.at[b], semL.at[b]).wait()

        start_load(0, 0)
        start_load(1, 1)

        def pair(k, carry):
            sdescs = []
            for b in range(2):
                i = k + b
                wait_load(i, b)
                sdescs.append(pltpu.async_copy(
                    mbuf.at[b], acc.at[ridx.at[b]], semS.at[b], add=True))
            for b in range(2):
                i = k + b
                sdescs[b].wait()
                nxt = jnp.minimum(i + 2, NCHUNK_S - 1)
                start_load(nxt, b)
            return carry

        lax.fori_loop(0, NCHUNK_S // 2, lambda k, c: pair(2 * k, c), 0)
        wait_load(NCHUNK_S - 1, 0)
        wait_load(NCHUNK_S - 1, 1)
        plsc.subcore_barrier()

        def _fin(nrows):
            pltpu.sync_copy(acc.at[pl.ds(rbase, nrows)], out_hbm.at[pl.ds(rbase, nrows)])

        @pl.when(s < NS - 1)
        def _f0():
            _fin(RPS)

        @pl.when(s == NS - 1)
        def _f1():
            _fin(RLAST)

    @pl.when(c == 0)
    def _c0():
        run(msgm_hbm, outm_hbm)

    @pl.when(c == 1)
    def _c1():
        run(msgp_hbm, outp_hbm)


def kernel(x, pos, pe, edge_index, batch,
           embed_W, embed_b, pe_W, pe_b,
           m1_W, m1_b, m2_W, m2_b,
           p1_W, p1_b, p2_W, p2_b,
           pr1_W, pr1_b, pr2_W, pr2_b,
           r1_W, r1_b, r2_W, r2_b):
    L = m1_W.shape[0]
    send = edge_index[0]
    rec = edge_index[1]
    pos16 = jnp.zeros((N, 16), _f32).at[:, :3].set(pos)
    zero_nd = jnp.zeros((N, D), _f32)
    batch3 = batch.astype(jnp.int32).reshape(N // BN, 1, BN)

    row = lambda v: v.reshape(1, -1)
    npad = EP - E
    pad_g = (jnp.arange(npad, dtype=jnp.int32) * 37) % N
    pad_s = N + (jnp.arange(npad, dtype=jnp.int32) % 8)
    send_p = jnp.concatenate([send, pad_g])
    rec_gp = jnp.concatenate([rec, pad_g])
    rec_sp = jnp.concatenate([rec, pad_s])
    sendA, sendB = send_p[:EH], send_p[EH:]
    recA, recB = rec_gp[:EH], rec_gp[EH:]
    recsA, recsB = rec_sp[:EH], rec_sp[EH:]
    hp = _embed(x, pe, embed_W[:D], embed_W[D:], row(embed_b), pe_W, row(pe_b))
    pos_s, pos_r = _pos_gather(send_p, rec_gp, pos16)
    dist = _dist(pos_s, pos_r)
    distA, distB = dist[:EH], dist[EH:]

    Z = jnp.zeros((D, D), _f32)
    agmA = agmB = agpA = agpB = zero_nd
    for l in range(L):
        ws = jnp.concatenate([
            jnp.concatenate([m1_W[l, 0:D], Z], axis=1),
            jnp.concatenate([m1_W[l, D:2 * D], p1_W[l, 0:D]], axis=1)], axis=0)
        wr = jnp.concatenate([
            jnp.concatenate([m1_W[l, 2 * D:3 * D], Z], axis=1),
            jnp.concatenate([m1_W[l, 3 * D:4 * D], p1_W[l, D:2 * D]], axis=1)], axis=0)
        bias = jnp.concatenate([m1_b[l], p1_b[l]]).reshape(1, D2)
        wd = jnp.concatenate([m1_W[l, 4 * D], p1_W[l, 2 * D]]).reshape(1, D2)

        hp, stab, rtab = _tables(hp, agmA, agmB, agpA, agpB, ws, wr, bias,
                                 1 if l == 0 else 2)
        preA = _edge_gather(sendA, recA, stab, rtab)
        msgmA, msgpA = _edge_mlp(preA, distA, wd,
                                 m2_W[l], row(m2_b[l]), p2_W[l], row(p2_b[l]))
        preB = _edge_gather(sendB, recB, stab, rtab)
        msgmB, msgpB = _edge_mlp(preB, distB, wd,
                                 m2_W[l], row(m2_b[l]), p2_W[l], row(p2_b[l]))
        agmA, agpA = _scatter_add(msgmA, msgpA, recsA, zero_nd)
        agmB, agpB = _scatter_add(msgmB, msgpB, recsB, zero_nd)

    r2p = jnp.zeros((D, D), _f32).at[:, :1].set(r2_W)
    rb2p = jnp.zeros((1, D), _f32).at[0, 0].set(r2_b[0])
    out = _readout(hp, agmA, agmB, batch3, pr1_W, row(pr1_b), pr2_W, row(pr2_b),
                   r1_W, row(r1_b), r2p, rb2p)
    return out[:, 0]


# speedup vs baseline: 1.2185x; 1.0896x over previous
import functools

import jax
import jax.numpy as jnp
from jax import lax
from jax.experimental import pallas as pl
from jax.experimental.pallas import tpu as pltpu
from jax.experimental.pallas import tpu_sc as plsc

N = 10000
E = 320000
D = 128
D2 = 256
G = 64

NC = 2
NS = 16
NW = NC * NS
EP = 327680
EH = EP // 2
EPW = EP // NW
EC = 128
NCHUNK = EPW // EC
EPWH = EH // NW
ECH = 80
NCHUNKH = EPWH // ECH
BN = 1000
BE = 640
RPS = 632
RLAST = N - RPS * (NS - 1)
NACC = N + 8
EPS = EH // NS
ECS = 128
NCHUNK_S = EPS // ECS

_f32 = jnp.float32
_mesh = plsc.VectorSubcoreMesh(core_axis_name="c", subcore_axis_name="s",
                               num_cores=NC, num_subcores=NS)


def _embed_body(x_ref, pe_ref, wx_ref, wpe_ref, eb_ref, pw_ref, pb_ref, hp_ref):
    x = x_ref[...]
    pe = pe_ref[...]
    h = (jnp.dot(x, wx_ref[...], preferred_element_type=_f32, precision=lax.Precision.HIGHEST)
         + jnp.dot(pe, wpe_ref[...], preferred_element_type=_f32, precision=lax.Precision.HIGHEST) + eb_ref[...])
    p = jnp.dot(pe, pw_ref[...], preferred_element_type=_f32, precision=lax.Precision.HIGHEST) + pb_ref[...]
    hp_ref[...] = jnp.concatenate([h, p], axis=1)


def _embed(x, pe, wx, wpe, eb, pw, pb):
    nb = N // BN
    return pl.pallas_call(
        _embed_body,
        grid=(nb,),
        in_specs=[
            pl.BlockSpec((BN, D), lambda i: (i, 0)),
            pl.BlockSpec((BN, 24), lambda i: (i, 0)),
            pl.BlockSpec((D, D), lambda i: (0, 0)),
            pl.BlockSpec((24, D), lambda i: (0, 0)),
            pl.BlockSpec((1, D), lambda i: (0, 0)),
            pl.BlockSpec((24, D), lambda i: (0, 0)),
            pl.BlockSpec((1, D), lambda i: (0, 0)),
        ],
        out_specs=pl.BlockSpec((BN, D2), lambda i: (i, 0)),
        out_shape=jax.ShapeDtypeStruct((N, D2), _f32),
    )(x, pe, wx, wpe, eb, pw, pb)


def _tables_body(s, hp_ref, agma_ref, agmb_ref, agpa_ref, agpb_ref,
                 ws_ref, wr_ref, b_ref, hpn_ref, send_ref, rec_ref):
    hp = hp_ref[...]
    ag = jnp.concatenate([agma_ref[...] + agmb_ref[...],
                          agpa_ref[...] + agpb_ref[...]], axis=1)
    hpn = s * hp + ag
    hpn_ref[...] = hpn
    send_ref[...] = jnp.dot(hpn, ws_ref[...], preferred_element_type=_f32, precision=lax.Precision.HIGHEST) + b_ref[...]
    rec_ref[...] = jnp.dot(hpn, wr_ref[...], preferred_element_type=_f32, precision=lax.Precision.HIGHEST)


def _tables(hp, agma, agmb, agpa, agpb, ws, wr, b, s):
    nb = N // BN
    return pl.pallas_call(
        functools.partial(_tables_body, float(s)),
        grid=(nb,),
        in_specs=[
            pl.BlockSpec((BN, D2), lambda i: (i, 0)),
            pl.BlockSpec((BN, D), lambda i: (i, 0)),
            pl.BlockSpec((BN, D), lambda i: (i, 0)),
            pl.BlockSpec((BN, D), lambda i: (i, 0)),
            pl.BlockSpec((BN, D), lambda i: (i, 0)),
            pl.BlockSpec((D2, D2), lambda i: (0, 0)),
            pl.BlockSpec((D2, D2), lambda i: (0, 0)),
            pl.BlockSpec((1, D2), lambda i: (0, 0)),
        ],
        out_specs=[
            pl.BlockSpec((BN, D2), lambda i: (i, 0)),
            pl.BlockSpec((BN, D2), lambda i: (i, 0)),
            pl.BlockSpec((BN, D2), lambda i: (i, 0)),
        ],
        out_shape=[
            jax.ShapeDtypeStruct((N, D2), _f32),
            jax.ShapeDtypeStruct((N, D2), _f32),
            jax.ShapeDtypeStruct((N, D2), _f32),
        ],
    )(hp, agma, agmb, agpa, agpb, ws, wr, b)


def _silu(z):
    return z * jax.nn.sigmoid(z)


def _dist_body(qs_ref, qr_ref, dist_ref):
    diff = qs_ref[...] - qr_ref[...]
    dist_ref[...] = jnp.sqrt(jnp.sum(diff * diff, axis=1, keepdims=True) + 1e-12)


def _dist(pos_s, pos_r):
    nb = EP // BE
    return pl.pallas_call(
        _dist_body,
        grid=(nb,),
        in_specs=[
            pl.BlockSpec((BE, 16), lambda i: (i, 0)),
            pl.BlockSpec((BE, 16), lambda i: (i, 0)),
        ],
        out_specs=pl.BlockSpec((BE, 1), lambda i: (i, 0)),
        out_shape=jax.ShapeDtypeStruct((EP, 1), _f32),
    )(pos_s, pos_r)


def _edge_body(ps_ref, dist_ref, wd_ref, m2w_ref, m2b_ref,
               p2w_ref, p2b_ref, msgm_ref, msgp_ref):
    pre = ps_ref[...] + dist_ref[...] * wd_ref[...]
    u = _silu(pre[:, :D])
    v = jnp.tanh(pre[:, D:])
    msgm_ref[...] = _silu(jnp.dot(u, m2w_ref[...], preferred_element_type=_f32)
                          + m2b_ref[...])
    msgp_ref[...] = jnp.tanh(jnp.dot(v, p2w_ref[...], preferred_element_type=_f32)
                             + p2b_ref[...])


def _edge_mlp(preSR, dist, wd, m2w, m2b, p2w, p2b):
    nb = EH // BE
    return pl.pallas_call(
        _edge_body,
        grid=(nb,),
        in_specs=[
            pl.BlockSpec((BE, D2), lambda i: (i, 0)),
            pl.BlockSpec((BE, 1), lambda i: (i, 0)),
            pl.BlockSpec((1, D2), lambda i: (0, 0)),
            pl.BlockSpec((D, D), lambda i: (0, 0)),
            pl.BlockSpec((1, D), lambda i: (0, 0)),
            pl.BlockSpec((D, D), lambda i: (0, 0)),
            pl.BlockSpec((1, D), lambda i: (0, 0)),
        ],
        out_specs=[
            pl.BlockSpec((BE, D), lambda i: (i, 0)),
            pl.BlockSpec((BE, D), lambda i: (i, 0)),
        ],
        out_shape=[
            jax.ShapeDtypeStruct((EH, D), _f32),
            jax.ShapeDtypeStruct((EH, D), _f32),
        ],
    )(preSR, dist, wd, m2w, m2b, p2w, p2b)


def _readout_body(hp_ref, agma_ref, agmb_ref, b3_ref, w1_ref, b1_ref, w2_ref,
                  b2_ref, r1_ref, rb1_ref, r2_ref, rb2_ref, out_ref, pool_ref):
    i = pl.program_id(0)
    nb = pl.num_programs(0)

    @pl.when(i == 0)
    def _init():
        pool_ref[...] = jnp.zeros_like(pool_ref)

    h = 2.0 * hp_ref[:, :D] + agma_ref[...] + agmb_ref[...]
    q = _silu(jnp.dot(h, w1_ref[...], preferred_element_type=_f32, precision=lax.Precision.HIGHEST) + b1_ref[...])
    q = jnp.dot(q, w2_ref[...], preferred_element_type=_f32, precision=lax.Precision.HIGHEST) + b2_ref[...]
    brow = b3_ref[0]
    gid = lax.broadcasted_iota(jnp.int32, (G, BN), 0)
    onehot = (brow == gid).astype(_f32)
    pool_ref[...] += jnp.dot(onehot, q, preferred_element_type=_f32, precision=lax.Precision.HIGHEST)

    @pl.when(i == nb - 1)
    def _fin():
        g = pool_ref[...]
        o = _silu(jnp.dot(g, r1_ref[...], preferred_element_type=_f32, precision=lax.Precision.HIGHEST) + rb1_ref[...])
        out_ref[...] = jnp.dot(o, r2_ref[...], preferred_element_type=_f32, precision=lax.Precision.HIGHEST) + rb2_ref[...]


def _readout(hp, agma, agmb, batch3, w1, b1, w2, b2, r1, rb1, r2p, rb2p):
    nb = N // BN
    return pl.pallas_call(
        _readout_body,
        grid=(nb,),
        in_specs=[
            pl.BlockSpec((BN, D2), lambda i: (i, 0)),
            pl.BlockSpec((BN, D), lambda i: (i, 0)),
            pl.BlockSpec((BN, D), lambda i: (i, 0)),
            pl.BlockSpec((1, 1, BN), lambda i: (i, 0, 0)),
            pl.BlockSpec((D, D), lambda i: (0, 0)),
            pl.BlockSpec((1, D), lambda i: (0, 0)),
            pl.BlockSpec((D, D), lambda i: (0, 0)),
            pl.BlockSpec((1, D), lambda i: (0, 0)),
            pl.BlockSpec((D, D), lambda i: (0, 0)),
            pl.BlockSpec((1, D), lambda i: (0, 0)),
            pl.BlockSpec((D, D), lambda i: (0, 0)),
            pl.BlockSpec((1, D), lambda i: (0, 0)),
        ],
        out_specs=pl.BlockSpec((G, D), lambda i: (0, 0)),
        out_shape=jax.ShapeDtypeStruct((G, D), _f32),
        scratch_shapes=[pltpu.VMEM((G, D), _f32)],
    )(hp, agma, agmb, batch3, w1, b1, w2, b2, r1, rb1, r2p, rb2p)


@functools.partial(
    pl.kernel,
    out_type=[
        jax.ShapeDtypeStruct((EP, 16), _f32),
        jax.ShapeDtypeStruct((EP, 16), _f32),
    ],
    mesh=_mesh,
    scratch_types=[
        pltpu.VMEM((EC,), jnp.int32),
        pltpu.VMEM((EC,), jnp.int32),
        pltpu.VMEM((EC, 16), _f32),
        pltpu.VMEM((EC, 16), _f32),
        pltpu.SemaphoreType.DMA,
        pltpu.SemaphoreType.DMA,
    ],
    compiler_params=pltpu.CompilerParams(use_tc_tiling_on_sc=False),
)
def _pos_gather(send_hbm, rec_hbm, pos_hbm, outs_hbm, outr_hbm,
                sidx, ridx, bufs, bufr, sems, semr):
    wid = lax.axis_index("s") * NC + lax.axis_index("c")
    base = wid * EPW

    def body(i, carry):
        off = pl.multiple_of(base + i * EC, 8)
        pltpu.sync_copy(send_hbm.at[pl.ds(off, EC)], sidx)
        pltpu.sync_copy(rec_hbm.at[pl.ds(off, EC)], ridx)
        cs = pltpu.async_copy(pos_hbm.at[sidx], bufs, sems)
        cr = pltpu.async_copy(pos_hbm.at[ridx], bufr, semr)
        cs.wait()
        cr.wait()
        pltpu.sync_copy(bufs, outs_hbm.at[pl.ds(off, EC)])
        pltpu.sync_copy(bufr, outr_hbm.at[pl.ds(off, EC)])
        return carry

    lax.fori_loop(0, NCHUNK, body, 0)


@functools.partial(
    pl.kernel,
    out_type=jax.ShapeDtypeStruct((EH, D2), _f32),
    mesh=_mesh,
    scratch_types=[
        pltpu.VMEM((2, ECH), jnp.int32),
        pltpu.VMEM((2, ECH), jnp.int32),
        pltpu.VMEM((2, ECH, D2), _f32),
        pltpu.VMEM((2, ECH, D2), _f32),
        pltpu.SemaphoreType.DMA((2,)),
        pltpu.SemaphoreType.DMA((2,)),
        pltpu.SemaphoreType.DMA((2,)),
    ],
)
def _edge_gather(send_hbm, rec_hbm, stab_hbm, rtab_hbm, outs_hbm,
                 sidx, ridx, bufS, bufR, semi, semg, semw):
    wid = lax.axis_index("s") * NC + lax.axis_index("c")
    base = wid * EPWH

    def idx_off(i):
        return pl.multiple_of(base + i * ECH, 8)

    def start_idx(i, b):
        off = idx_off(i)
        pltpu.async_copy(send_hbm.at[pl.ds(off, ECH)], sidx.at[b], semi.at[b])
        pltpu.async_copy(rec_hbm.at[pl.ds(off, ECH)], ridx.at[b], semi.at[b])

    def wait_idx(i, b):
        off = idx_off(i)
        pltpu.make_async_copy(send_hbm.at[pl.ds(off, ECH)], sidx.at[b], semi.at[b]).wait()
        pltpu.make_async_copy(rec_hbm.at[pl.ds(off, ECH)], ridx.at[b], semi.at[b]).wait()

    start_idx(0, 0)
    start_idx(1, 1)

    def pair(k, carry):
        descs = []
        for b in range(2):
            i = k + b
            wait_idx(i, b)
            descs.append(
                (pltpu.async_copy(stab_hbm.at[sidx.at[b]], bufS.at[b], semg.at[b]),
                 pltpu.async_copy(rtab_hbm.at[ridx.at[b]], bufR.at[b], semg.at[b])))
        wdescs = []
        for b in range(2):
            i = k + b
            descs[b][0].wait()
            descs[b][1].wait()
            nxt = jnp.minimum(i + 2, NCHUNKH - 1)
            start_idx(nxt, b)
            bs2 = bufS.at[b]
            br2 = bufR.at[b]

            def add_row(r, carry2, bs2=bs2, br2=br2):
                for kk in range(D2 // 16):
                    sl = pl.ds(kk * 16, 16)
                    bs2[r, sl] = bs2[r, sl] + br2[r, sl]
                return carry2

            lax.fori_loop(0, ECH, add_row, 0)
            off = idx_off(i)
            wdescs.append(
                pltpu.async_copy(bufS.at[b], outs_hbm.at[pl.ds(off, ECH)], semw.at[b]))
        for b in range(2):
            wdescs[b].wait()
        return carry

    lax.fori_loop(0, NCHUNKH // 2, lambda k, c: pair(2 * k, c), 0)

    wait_idx(NCHUNKH - 1, 0)
    wait_idx(NCHUNKH - 1, 1)


@functools.partial(
    pl.kernel,
    out_type=[
        jax.ShapeDtypeStruct((N, D), _f32),
        jax.ShapeDtypeStruct((N, D), _f32),
    ],
    mesh=_mesh,
    scratch_types=[
        pltpu.VMEM((2, ECS), jnp.int32),
        pltpu.VMEM((2, ECS, D), _f32),
        pltpu.VMEM_SHARED((NACC, D), _f32),
        pltpu.SemaphoreType.DMA((2,)),
        pltpu.SemaphoreType.DMA((2,)),
    ],
)
def _scatter_add(msgm_hbm, msgp_hbm, rec_hbm, zero_hbm, outm_hbm, outp_hbm,
                 ridx, mbuf, acc, semL, semS):
    c = lax.axis_index("c")
    s = lax.axis_index("s")
    rbase = pl.multiple_of(s * RPS, 8)

    def _init(nrows):
        pltpu.sync_copy(zero_hbm.at[pl.ds(rbase, nrows)], acc.at[pl.ds(rbase, nrows)])

    @pl.when(s < NS - 1)
    def _i0():
        _init(RPS)

    @pl.when(s == NS - 1)
    def _i1():
        _init(RLAST)

    plsc.subcore_barrier()

    def run(msg_hbm, out_hbm):
        ebase = s * EPS

        def chunk_off(i):
            return pl.multiple_of(ebase + i * ECS, 8)

        def start_load(i, b):
            off = chunk_off(i)
            pltpu.async_copy(rec_hbm.at[pl.ds(off, ECS)], ridx.at[b], semL.at[b])
            pltpu.async_copy(msg_hbm.at[pl.ds(off, ECS)], mbuf.at[b], semL.at[b])

        def wait_load(i, b):
            off = chunk_off(i)
            pltpu.make_async_copy(rec_hbm.at[pl.ds(off, ECS)], ridx.at[b], semL.at[b]).wait()
            pltpu.make_async_copy(msg_hbm.at[pl.ds(off, ECS)], mbuf.at[b], semL.at[b]).wait()

        start_load(0, 0)
        start_load(1, 1)

        def pair(k, carry):
            sdescs = []
            for b in range(2):
                i = k + b
                wait_load(i, b)
                sdescs.append(pltpu.async_copy(
                    mbuf.at[b], acc.at[ridx.at[b]], semS.at[b], add=True))
            for b in range(2):
                i = k + b
                sdescs[b].wait()
                nxt = jnp.minimum(i + 2, NCHUNK_S - 1)
                start_load(nxt, b)
            return carry

        lax.fori_loop(0, NCHUNK_S // 2, lambda k, c: pair(2 * k, c), 0)
        wait_load(NCHUNK_S - 1, 0)
        wait_load(NCHUNK_S - 1, 1)
        plsc.subcore_barrier()

        def _fin(nrows):
            pltpu.sync_copy(acc.at[pl.ds(rbase, nrows)], out_hbm.at[pl.ds(rbase, nrows)])

        @pl.when(s < NS - 1)
        def _f0():
            _fin(RPS)

        @pl.when(s == NS - 1)
        def _f1():
            _fin(RLAST)

    @pl.when(c == 0)
    def _c0():
        run(msgm_hbm, outm_hbm)

    @pl.when(c == 1)
    def _c1():
        run(msgp_hbm, outp_hbm)


def kernel(x, pos, pe, edge_index, batch,
           embed_W, embed_b, pe_W, pe_b,
           m1_W, m1_b, m2_W, m2_b,
           p1_W, p1_b, p2_W, p2_b,
           pr1_W, pr1_b, pr2_W, pr2_b,
           r1_W, r1_b, r2_W, r2_b):
    L = m1_W.shape[0]
    send = edge_index[0]
    rec = edge_index[1]
    pos16 = jnp.zeros((N, 16), _f32).at[:, :3].set(pos)
    zero_nd = jnp.zeros((N, D), _f32)
    batch3 = batch.astype(jnp.int32).reshape(N // BN, 1, BN)

    row = lambda v: v.reshape(1, -1)
    npad = EP - E
    pad_g = (jnp.arange(npad, dtype=jnp.int32) * 37) % N
    pad_s = N + (jnp.arange(npad, dtype=jnp.int32) % 8)
    send_p = jnp.concatenate([send, pad_g])
    rec_gp = jnp.concatenate([rec, pad_g])
    rec_sp = jnp.concatenate([rec, pad_s])
    sendA, sendB = send_p[:EH], send_p[EH:]
    recA, recB = rec_gp[:EH], rec_gp[EH:]
    recsA, recsB = rec_sp[:EH], rec_sp[EH:]
    hp = _embed(x, pe, embed_W[:D], embed_W[D:], row(embed_b), pe_W, row(pe_b))
    pos_s, pos_r = _pos_gather(send_p, rec_gp, pos16)
    dist = _dist(pos_s, pos_r)
    distA, distB = dist[:EH], dist[EH:]

    Z = jnp.zeros((D, D), _f32)
    agmA = agmB = agpA = agpB = zero_nd
    for l in range(L):
        ws = jnp.concatenate([
            jnp.concatenate([m1_W[l, 0:D], Z], axis=1),
            jnp.concatenate([m1_W[l, D:2 * D], p1_W[l, 0:D]], axis=1)], axis=0)
        wr = jnp.concatenate([
            jnp.concatenate([m1_W[l, 2 * D:3 * D], Z], axis=1),
            jnp.concatenate([m1_W[l, 3 * D:4 * D], p1_W[l, D:2 * D]], axis=1)], axis=0)
        bias = jnp.concatenate([m1_b[l], p1_b[l]]).reshape(1, D2)
        wd = jnp.concatenate([m1_W[l, 4 * D], p1_W[l, 2 * D]]).reshape(1, D2)

        hp, stab, rtab = _tables(hp, agmA, agmB, agpA, agpB, ws, wr, bias,
                                 1 if l == 0 else 2)
        preA = _edge_gather(sendA, recA, stab, rtab)
        msgmA, msgpA = _edge_mlp(preA, distA, wd,
                                 m2_W[l], row(m2_b[l]), p2_W[l], row(p2_b[l]))
        preB = _edge_gather(sendB, recB, stab, rtab)
        msgmB, msgpB = _edge_mlp(preB, distB, wd,
                                 m2_W[l], row(m2_b[l]), p2_W[l], row(p2_b[l]))
        agmA, agpA = _scatter_add(msgmA, msgpA, recsA, zero_nd)
        agmB, agpB = _scatter_add(msgmB, msgpB, recsB, zero_nd)

    r2p = jnp.zeros((D, D), _f32).at[:, :1].set(r2_W)
    rb2p = jnp.zeros((1, D), _f32).at[0, 0].set(r2_b[0])
    out = _readout(hp, agmA, agmB, batch3, pr1_W, row(pr1_b), pr2_W, row(pr2_b),
                   r1_W, row(r1_b), r2p, rb2p)
    return out[:, 0]


# speedup vs baseline: 1.2778x; 1.0487x over previous
import functools

import jax
import jax.numpy as jnp
from jax import lax
from jax.experimental import pallas as pl
from jax.experimental.pallas import tpu as pltpu
from jax.experimental.pallas import tpu_sc as plsc

N = 10000
E = 320000
D = 128
D2 = 256
G = 64

NC = 2
NS = 16
NW = NC * NS
EP = 327680
EH = EP // 2
EPW = EP // NW
EC = 128
NCHUNK = EPW // EC
EPWH = EH // NW
ECH = 80
NCHUNKH = EPWH // ECH
BN = 1000
BE = 640
RPS = 632
RLAST = N - RPS * (NS - 1)
NACC = N + 8
EPS = EH // NS
ECS = 128
NCHUNK_S = EPS // ECS

_f32 = jnp.float32
_mesh = plsc.VectorSubcoreMesh(core_axis_name="c", subcore_axis_name="s",
                               num_cores=NC, num_subcores=NS)


def _embed_body(x_ref, pe_ref, wx_ref, wpe_ref, eb_ref, pw_ref, pb_ref, hp_ref):
    x = x_ref[...]
    pe = pe_ref[...]
    h = (jnp.dot(x, wx_ref[...], preferred_element_type=_f32, precision=lax.Precision.HIGHEST)
         + jnp.dot(pe, wpe_ref[...], preferred_element_type=_f32, precision=lax.Precision.HIGHEST) + eb_ref[...])
    p = jnp.dot(pe, pw_ref[...], preferred_element_type=_f32, precision=lax.Precision.HIGHEST) + pb_ref[...]
    hp_ref[...] = jnp.concatenate([h, p], axis=1)


def _embed(x, pe, wx, wpe, eb, pw, pb):
    nb = N // BN
    return pl.pallas_call(
        _embed_body,
        grid=(nb,),
        in_specs=[
            pl.BlockSpec((BN, D), lambda i: (i, 0)),
            pl.BlockSpec((BN, 24), lambda i: (i, 0)),
            pl.BlockSpec((D, D), lambda i: (0, 0)),
            pl.BlockSpec((24, D), lambda i: (0, 0)),
            pl.BlockSpec((1, D), lambda i: (0, 0)),
            pl.BlockSpec((24, D), lambda i: (0, 0)),
            pl.BlockSpec((1, D), lambda i: (0, 0)),
        ],
        out_specs=pl.BlockSpec((BN, D2), lambda i: (i, 0)),
        out_shape=jax.ShapeDtypeStruct((N, D2), _f32),
    )(x, pe, wx, wpe, eb, pw, pb)


def _tables_body(s, hp_ref, agma_ref, agmb_ref, agpa_ref, agpb_ref,
                 ws_ref, wr_ref, b_ref, hpn_ref, send_ref, rec_ref):
    hp = hp_ref[...]
    ag = jnp.concatenate([agma_ref[...] + agmb_ref[...],
                          agpa_ref[...] + agpb_ref[...]], axis=1)
    hpn = s * hp + ag
    hpn_ref[...] = hpn
    send_ref[...] = jnp.dot(hpn, ws_ref[...], preferred_element_type=_f32, precision=lax.Precision.HIGHEST) + b_ref[...]
    rec_ref[...] = jnp.dot(hpn, wr_ref[...], preferred_element_type=_f32, precision=lax.Precision.HIGHEST)


def _tables(hp, agma, agmb, agpa, agpb, ws, wr, b, s):
    nb = N // BN
    return pl.pallas_call(
        functools.partial(_tables_body, float(s)),
        grid=(nb,),
        in_specs=[
            pl.BlockSpec((BN, D2), lambda i: (i, 0)),
            pl.BlockSpec((BN, D), lambda i: (i, 0)),
            pl.BlockSpec((BN, D), lambda i: (i, 0)),
            pl.BlockSpec((BN, D), lambda i: (i, 0)),
            pl.BlockSpec((BN, D), lambda i: (i, 0)),
            pl.BlockSpec((D2, D2), lambda i: (0, 0)),
            pl.BlockSpec((D2, D2), lambda i: (0, 0)),
            pl.BlockSpec((1, D2), lambda i: (0, 0)),
        ],
        out_specs=[
            pl.BlockSpec((BN, D2), lambda i: (i, 0)),
            pl.BlockSpec((BN, D2), lambda i: (i, 0)),
            pl.BlockSpec((BN, D2), lambda i: (i, 0)),
        ],
        out_shape=[
            jax.ShapeDtypeStruct((N, D2), _f32),
            jax.ShapeDtypeStruct((N, D2), _f32),
            jax.ShapeDtypeStruct((N, D2), _f32),
        ],
    )(hp, agma, agmb, agpa, agpb, ws, wr, b)


def _silu(z):
    return z * jax.nn.sigmoid(z)


def _dist_body(qd_ref, dist_ref):
    diff = qd_ref[...]
    dist_ref[...] = jnp.sqrt(jnp.sum(diff * diff, axis=1, keepdims=True) + 1e-12)


def _dist(pos_d):
    nb = EP // BE
    return pl.pallas_call(
        _dist_body,
        grid=(nb,),
        in_specs=[
            pl.BlockSpec((BE, 16), lambda i: (i, 0)),
        ],
        out_specs=pl.BlockSpec((BE, 1), lambda i: (i, 0)),
        out_shape=jax.ShapeDtypeStruct((EP, 1), _f32),
    )(pos_d)


def _edge_body(ps_ref, dist_ref, wd_ref, m2w_ref, m2b_ref,
               p2w_ref, p2b_ref, msgm_ref, msgp_ref):
    pre = ps_ref[...] + dist_ref[...] * wd_ref[...]
    u = _silu(pre[:, :D])
    v = jnp.tanh(pre[:, D:])
    msgm_ref[...] = _silu(jnp.dot(u, m2w_ref[...], preferred_element_type=_f32)
                          + m2b_ref[...])
    msgp_ref[...] = jnp.tanh(jnp.dot(v, p2w_ref[...], preferred_element_type=_f32)
                             + p2b_ref[...])


def _edge_mlp(preSR, dist, wd, m2w, m2b, p2w, p2b):
    nb = EH // BE
    return pl.pallas_call(
        _edge_body,
        grid=(nb,),
        in_specs=[
            pl.BlockSpec((BE, D2), lambda i: (i, 0)),
            pl.BlockSpec((BE, 1), lambda i: (i, 0)),
            pl.BlockSpec((1, D2), lambda i: (0, 0)),
            pl.BlockSpec((D, D), lambda i: (0, 0)),
            pl.BlockSpec((1, D), lambda i: (0, 0)),
            pl.BlockSpec((D, D), lambda i: (0, 0)),
            pl.BlockSpec((1, D), lambda i: (0, 0)),
        ],
        out_specs=[
            pl.BlockSpec((BE, D), lambda i: (i, 0)),
            pl.BlockSpec((BE, D), lambda i: (i, 0)),
        ],
        out_shape=[
            jax.ShapeDtypeStruct((EH, D), _f32),
            jax.ShapeDtypeStruct((EH, D), _f32),
        ],
    )(preSR, dist, wd, m2w, m2b, p2w, p2b)


def _readout_body(hp_ref, agma_ref, agmb_ref, b3_ref, w1_ref, b1_ref, w2_ref,
                  b2_ref, r1_ref, rb1_ref, r2_ref, rb2_ref, out_ref, pool_ref):
    i = pl.program_id(0)
    nb = pl.num_programs(0)

    @pl.when(i == 0)
    def _init():
        pool_ref[...] = jnp.zeros_like(pool_ref)

    h = 2.0 * hp_ref[:, :D] + agma_ref[...] + agmb_ref[...]
    q = _silu(jnp.dot(h, w1_ref[...], preferred_element_type=_f32, precision=lax.Precision.HIGHEST) + b1_ref[...])
    q = jnp.dot(q, w2_ref[...], preferred_element_type=_f32, precision=lax.Precision.HIGHEST) + b2_ref[...]
    brow = b3_ref[0]
    gid = lax.broadcasted_iota(jnp.int32, (G, BN), 0)
    onehot = (brow == gid).astype(_f32)
    pool_ref[...] += jnp.dot(onehot, q, preferred_element_type=_f32, precision=lax.Precision.HIGHEST)

    @pl.when(i == nb - 1)
    def _fin():
        g = pool_ref[...]
        o = _silu(jnp.dot(g, r1_ref[...], preferred_element_type=_f32, precision=lax.Precision.HIGHEST) + rb1_ref[...])
        out_ref[...] = jnp.dot(o, r2_ref[...], preferred_element_type=_f32, precision=lax.Precision.HIGHEST) + rb2_ref[...]


def _readout(hp, agma, agmb, batch3, w1, b1, w2, b2, r1, rb1, r2p, rb2p):
    nb = N // BN
    return pl.pallas_call(
        _readout_body,
        grid=(nb,),
        in_specs=[
            pl.BlockSpec((BN, D2), lambda i: (i, 0)),
            pl.BlockSpec((BN, D), lambda i: (i, 0)),
            pl.BlockSpec((BN, D), lambda i: (i, 0)),
            pl.BlockSpec((1, 1, BN), lambda i: (i, 0, 0)),
            pl.BlockSpec((D, D), lambda i: (0, 0)),
            pl.BlockSpec((1, D), lambda i: (0, 0)),
            pl.BlockSpec((D, D), lambda i: (0, 0)),
            pl.BlockSpec((1, D), lambda i: (0, 0)),
            pl.BlockSpec((D, D), lambda i: (0, 0)),
            pl.BlockSpec((1, D), lambda i: (0, 0)),
            pl.BlockSpec((D, D), lambda i: (0, 0)),
            pl.BlockSpec((1, D), lambda i: (0, 0)),
        ],
        out_specs=pl.BlockSpec((G, D), lambda i: (0, 0)),
        out_shape=jax.ShapeDtypeStruct((G, D), _f32),
        scratch_shapes=[pltpu.VMEM((G, D), _f32)],
    )(hp, agma, agmb, batch3, w1, b1, w2, b2, r1, rb1, r2p, rb2p)


@functools.partial(
    pl.kernel,
    out_type=jax.ShapeDtypeStruct((EP, 16), _f32),
    mesh=_mesh,
    scratch_types=[
        pltpu.VMEM((2, EC), jnp.int32),
        pltpu.VMEM((2, EC), jnp.int32),
        pltpu.VMEM((2, EC, 16), _f32),
        pltpu.VMEM((2, EC, 16), _f32),
        pltpu.SemaphoreType.DMA((2,)),
        pltpu.SemaphoreType.DMA((2,)),
        pltpu.SemaphoreType.DMA((2,)),
    ],
    compiler_params=pltpu.CompilerParams(use_tc_tiling_on_sc=False),
)
def _pos_gather(send_hbm, rec_hbm, pos_hbm, outd_hbm,
                sidx, ridx, bufS, bufR, semi, semg, semw):
    wid = lax.axis_index("s") * NC + lax.axis_index("c")
    base = wid * EPW

    def idx_off(i):
        return pl.multiple_of(base + i * EC, 8)

    def start_idx(i, b):
        off = idx_off(i)
        pltpu.async_copy(send_hbm.at[pl.ds(off, EC)], sidx.at[b], semi.at[b])
        pltpu.async_copy(rec_hbm.at[pl.ds(off, EC)], ridx.at[b], semi.at[b])

    def wait_idx(i, b):
        off = idx_off(i)
        pltpu.make_async_copy(send_hbm.at[pl.ds(off, EC)], sidx.at[b], semi.at[b]).wait()
        pltpu.make_async_copy(rec_hbm.at[pl.ds(off, EC)], ridx.at[b], semi.at[b]).wait()

    start_idx(0, 0)
    start_idx(1, 1)

    def pair(k, carry):
        descs = []
        for b in range(2):
            i = k + b
            wait_idx(i, b)
            descs.append(
                (pltpu.async_copy(pos_hbm.at[sidx.at[b]], bufS.at[b], semg.at[b]),
                 pltpu.async_copy(pos_hbm.at[ridx.at[b]], bufR.at[b], semg.at[b])))
        wdescs = []
        for b in range(2):
            i = k + b
            descs[b][0].wait()
            descs[b][1].wait()
            nxt = jnp.minimum(i + 2, NCHUNK - 1)
            start_idx(nxt, b)
            bs2 = bufS.at[b]
            br2 = bufR.at[b]

            def sub_row(r, carry2, bs2=bs2, br2=br2):
                sl = pl.ds(0, 16)
                bs2[r, sl] = bs2[r, sl] - br2[r, sl]
                return carry2

            lax.fori_loop(0, EC, sub_row, 0)
            off = idx_off(i)
            wdescs.append(
                pltpu.async_copy(bufS.at[b], outd_hbm.at[pl.ds(off, EC)], semw.at[b]))
        for b in range(2):
            wdescs[b].wait()
        return carry

    lax.fori_loop(0, NCHUNK // 2, lambda k, c: pair(2 * k, c), 0)
    wait_idx(NCHUNK - 1, 0)
    wait_idx(NCHUNK - 1, 1)


@functools.partial(
    pl.kernel,
    out_type=jax.ShapeDtypeStruct((EH, D2), _f32),
    mesh=_mesh,
    scratch_types=[
        pltpu.VMEM((2, ECH), jnp.int32),
        pltpu.VMEM((2, ECH), jnp.int32),
        pltpu.VMEM((2, ECH, D2), _f32),
        pltpu.VMEM((2, ECH, D2), _f32),
        pltpu.SemaphoreType.DMA((2,)),
        pltpu.SemaphoreType.DMA((2,)),
        pltpu.SemaphoreType.DMA((2,)),
    ],
)
def _edge_gather(send_hbm, rec_hbm, stab_hbm, rtab_hbm, outs_hbm,
                 sidx, ridx, bufS, bufR, semi, semg, semw):
    wid = lax.axis_index("s") * NC + lax.axis_index("c")
    base = wid * EPWH

    def idx_off(i):
        return pl.multiple_of(base + i * ECH, 8)

    def start_idx(i, b):
        off = idx_off(i)
        pltpu.async_copy(send_hbm.at[pl.ds(off, ECH)], sidx.at[b], semi.at[b])
        pltpu.async_copy(rec_hbm.at[pl.ds(off, ECH)], ridx.at[b], semi.at[b])

    def wait_idx(i, b):
        off = idx_off(i)
        pltpu.make_async_copy(send_hbm.at[pl.ds(off, ECH)], sidx.at[b], semi.at[b]).wait()
        pltpu.make_async_copy(rec_hbm.at[pl.ds(off, ECH)], ridx.at[b], semi.at[b]).wait()

    start_idx(0, 0)
    start_idx(1, 1)

    def pair(k, carry):
        descs = []
        for b in range(2):
            i = k + b
            wait_idx(i, b)
            descs.append(
                (pltpu.async_copy(stab_hbm.at[sidx.at[b]], bufS.at[b], semg.at[b]),
                 pltpu.async_copy(rtab_hbm.at[ridx.at[b]], bufR.at[b], semg.at[b])))
        wdescs = []
        for b in range(2):
            i = k + b
            descs[b][0].wait()
            descs[b][1].wait()
            nxt = jnp.minimum(i + 2, NCHUNKH - 1)
            start_idx(nxt, b)
            bs2 = bufS.at[b]
            br2 = bufR.at[b]

            def add_row(r, carry2, bs2=bs2, br2=br2):
                for kk in range(D2 // 16):
                    sl = pl.ds(kk * 16, 16)
                    bs2[r, sl] = bs2[r, sl] + br2[r, sl]
                return carry2

            lax.fori_loop(0, ECH, add_row, 0)
            off = idx_off(i)
            wdescs.append(
                pltpu.async_copy(bufS.at[b], outs_hbm.at[pl.ds(off, ECH)], semw.at[b]))
        for b in range(2):
            wdescs[b].wait()
        return carry

    lax.fori_loop(0, NCHUNKH // 2, lambda k, c: pair(2 * k, c), 0)

    wait_idx(NCHUNKH - 1, 0)
    wait_idx(NCHUNKH - 1, 1)


@functools.partial(
    pl.kernel,
    out_type=[
        jax.ShapeDtypeStruct((N, D), _f32),
        jax.ShapeDtypeStruct((N, D), _f32),
    ],
    mesh=_mesh,
    scratch_types=[
        pltpu.VMEM((2, ECS), jnp.int32),
        pltpu.VMEM((2, ECS, D), _f32),
        pltpu.VMEM_SHARED((NACC, D), _f32),
        pltpu.SemaphoreType.DMA((2,)),
        pltpu.SemaphoreType.DMA((2,)),
    ],
)
def _scatter_add(msgm_hbm, msgp_hbm, rec_hbm, zero_hbm, outm_hbm, outp_hbm,
                 ridx, mbuf, acc, semL, semS):
    c = lax.axis_index("c")
    s = lax.axis_index("s")
    rbase = pl.multiple_of(s * RPS, 8)

    def _init(nrows):
        pltpu.sync_copy(zero_hbm.at[pl.ds(rbase, nrows)], acc.at[pl.ds(rbase, nrows)])

    @pl.when(s < NS - 1)
    def _i0():
        _init(RPS)

    @pl.when(s == NS - 1)
    def _i1():
        _init(RLAST)

    plsc.subcore_barrier()

    def run(msg_hbm, out_hbm):
        ebase = s * EPS

        def chunk_off(i):
            return pl.multiple_of(ebase + i * ECS, 8)

        def start_load(i, b):
            off = chunk_off(i)
            pltpu.async_copy(rec_hbm.at[pl.ds(off, ECS)], ridx.at[b], semL.at[b])
            pltpu.async_copy(msg_hbm.at[pl.ds(off, ECS)], mbuf.at[b], semL.at[b])

        def wait_load(i, b):
            off = chunk_off(i)
            pltpu.make_async_copy(rec_hbm.at[pl.ds(off, ECS)], ridx.at[b], semL.at[b]).wait()
            pltpu.make_async_copy(msg_hbm.at[pl.ds(off, ECS)], mbuf.at[b], semL.at[b]).wait()

        start_load(0, 0)
        start_load(1, 1)

        def pair(k, carry):
            sdescs = []
            for b in range(2):
                i = k + b
                wait_load(i, b)
                sdescs.append(pltpu.async_copy(
                    mbuf.at[b], acc.at[ridx.at[b]], semS.at[b], add=True))
            for b in range(2):
                i = k + b
                sdescs[b].wait()
                nxt = jnp.minimum(i + 2, NCHUNK_S - 1)
                start_load(nxt, b)
            return carry

        lax.fori_loop(0, NCHUNK_S // 2, lambda k, c: pair(2 * k, c), 0)
        wait_load(NCHUNK_S - 1, 0)
        wait_load(NCHUNK_S - 1, 1)
        plsc.subcore_barrier()

        def _fin(nrows):
            pltpu.sync_copy(acc.at[pl.ds(rbase, nrows)], out_hbm.at[pl.ds(rbase, nrows)])

        @pl.when(s < NS - 1)
        def _f0():
            _fin(RPS)

        @pl.when(s == NS - 1)
        def _f1():
            _fin(RLAST)

    @pl.when(c == 0)
    def _c0():
        run(msgm_hbm, outm_hbm)

    @pl.when(c == 1)
    def _c1():
        run(msgp_hbm, outp_hbm)


def kernel(x, pos, pe, edge_index, batch,
           embed_W, embed_b, pe_W, pe_b,
           m1_W, m1_b, m2_W, m2_b,
           p1_W, p1_b, p2_W, p2_b,
           pr1_W, pr1_b, pr2_W, pr2_b,
           r1_W, r1_b, r2_W, r2_b):
    L = m1_W.shape[0]
    send = edge_index[0]
    rec = edge_index[1]
    pos16 = jnp.zeros((N, 16), _f32).at[:, :3].set(pos)
    zero_nd = jnp.zeros((N, D), _f32)
    batch3 = batch.astype(jnp.int32).reshape(N // BN, 1, BN)

    row = lambda v: v.reshape(1, -1)
    npad = EP - E
    pad_g = (jnp.arange(npad, dtype=jnp.int32) * 37) % N
    pad_s = N + (jnp.arange(npad, dtype=jnp.int32) % 8)
    send_p = jnp.concatenate([send, pad_g])
    rec_gp = jnp.concatenate([rec, pad_g])
    rec_sp = jnp.concatenate([rec, pad_s])
    sendA, sendB = send_p[:EH], send_p[EH:]
    recA, recB = rec_gp[:EH], rec_gp[EH:]
    recsA, recsB = rec_sp[:EH], rec_sp[EH:]
    hp = _embed(x, pe, embed_W[:D], embed_W[D:], row(embed_b), pe_W, row(pe_b))
    pos_d = _pos_gather(send_p, rec_gp, pos16)
    dist = _dist(pos_d)
    distA, distB = dist[:EH], dist[EH:]

    Z = jnp.zeros((D, D), _f32)
    agmA = agmB = agpA = agpB = zero_nd
    for l in range(L):
        ws = jnp.concatenate([
            jnp.concatenate([m1_W[l, 0:D], Z], axis=1),
            jnp.concatenate([m1_W[l, D:2 * D], p1_W[l, 0:D]], axis=1)], axis=0)
        wr = jnp.concatenate([
            jnp.concatenate([m1_W[l, 2 * D:3 * D], Z], axis=1),
            jnp.concatenate([m1_W[l, 3 * D:4 * D], p1_W[l, D:2 * D]], axis=1)], axis=0)
        bias = jnp.concatenate([m1_b[l], p1_b[l]]).reshape(1, D2)
        wd = jnp.concatenate([m1_W[l, 4 * D], p1_W[l, 2 * D]]).reshape(1, D2)

        hp, stab, rtab = _tables(hp, agmA, agmB, agpA, agpB, ws, wr, bias,
                                 1 if l == 0 else 2)
        preA = _edge_gather(sendA, recA, stab, rtab)
        msgmA, msgpA = _edge_mlp(preA, distA, wd,
                                 m2_W[l], row(m2_b[l]), p2_W[l], row(p2_b[l]))
        preB = _edge_gather(sendB, recB, stab, rtab)
        msgmB, msgpB = _edge_mlp(preB, distB, wd,
                                 m2_W[l], row(m2_b[l]), p2_W[l], row(p2_b[l]))
        agmA, agpA = _scatter_add(msgmA, msgpA, recsA, zero_nd)
        agmB, agpB = _scatter_add(msgmB, msgpB, recsB, zero_nd)

    r2p = jnp.zeros((D, D), _f32).at[:, :1].set(r2_W)
    rb2p = jnp.zeros((1, D), _f32).at[0, 0].set(r2_b[0])
    out = _readout(hp, agmA, agmB, batch3, pr1_W, row(pr1_b), pr2_W, row(pr2_b),
                   r1_W, row(r1_b), r2p, rb2p)
    return out[:, 0]
